# Initial kernel scaffold; baseline (speedup 1.0000x reference)
import jax
import jax.numpy as jnp
from jax.experimental import pallas as pl


def kernel(h, v, f, pos, edge_index, edge_rbf, lin_msg_w, lin_msg_b, lin_vec_w, lin_vec_b, lin_scalar_w, lin_scalar_b, lin_edge_w, lin_edge_b, lin_angular_w, lin_angular_b, lin_dihedral_w, lin_dihedral_b):
    raise NotImplementedError("write your pallas kernel here")


# speedup vs baseline: 3.3906x; 3.3906x over previous
import functools

import jax
import jax.numpy as jnp
from jax import lax
from jax.experimental import pallas as pl

H = 256
R = 50
RP = 64
CUTOFF = 10.0

EB = 640
NB = 1000


def _geom_body(ev_ref, out_ref):
    ev = ev_ref[...]
    e0 = ev[:, 0:1]
    e1 = ev[:, 1:2]
    e2 = ev[:, 2:3]
    dist = jnp.sqrt(e0 * e0 + e1 * e1 + e2 * e2) + 1e-8
    inv = 1.0 / dist
    u0 = e0 * inv
    u1 = e1 * inv
    u2 = e2 * inv
    cw = 0.5 * (jnp.cos(jnp.pi * dist / CUTOFF) + 1.0)
    cw = jnp.where(dist < CUTOFF, cw, 0.0)
    out = jnp.concatenate([u0, u1, u2, cw, dist, dist, dist, dist], axis=1)
    out_ref[...] = out


def _edge_body(hc_ref, hr_ref, rbf_ref, vr_ref, f_ref, geom_ref, du2_ref,
               w1a_ref, w1b_ref, w1c_ref, b1_ref, wv_ref, bv_ref,
               we_ref, be_ref, wd_ref, bd_ref,
               msg_ref, fout_ref, dih_ref):
    hc = hc_ref[...]
    hr = hr_ref[...]
    rbf = rbf_ref[...]
    geom = geom_ref[...]
    u0 = geom[:, 0:1]
    u1 = geom[:, 1:2]
    u2 = geom[:, 2:3]
    cw = geom[:, 3:4]

    sm = (jnp.dot(hc, w1a_ref[...], preferred_element_type=jnp.float32)
          + jnp.dot(hr, w1b_ref[...], preferred_element_type=jnp.float32)
          + jnp.dot(rbf, w1c_ref[...], preferred_element_type=jnp.float32)
          + b1_ref[...])
    vw = jnp.dot(sm, wv_ref[...], preferred_element_type=jnp.float32) + bv_ref[...]
    w1 = vw[:, :H] * cw
    w2 = vw[:, H:] * cw
    vr = vr_ref[...]
    msg_ref[:, 0:H] = w1 * u0 + w2 * vr[:, 0:H]
    msg_ref[:, H:2 * H] = w1 * u1 + w2 * vr[:, H:2 * H]
    msg_ref[:, 2 * H:3 * H] = w1 * u2 + w2 * vr[:, 2 * H:3 * H]

    du2 = du2_ref[...]
    a0 = du2[:, 0:1]
    a1 = du2[:, 1:2]
    a2 = du2[:, 2:3]
    b0 = du2[:, 4:5]
    b1 = du2[:, 5:6]
    b2 = du2[:, 6:7]
    dvi = a0 * u0 + a1 * u1 + a2 * u2
    dvj = -(b0 * u0 + b1 * u1 + b2 * u2)
    w_ij0 = a0 - dvi * u0
    w_ij1 = a1 - dvi * u1
    w_ij2 = a2 - dvi * u2
    w_ji0 = b0 + dvj * u0
    w_ji1 = b1 + dvj * u1
    w_ji2 = b2 + dvj * u2
    dih = w_ij0 * w_ji0 + w_ij1 * w_ji1 + w_ij2 * w_ji2
    dih_b = jnp.broadcast_to(dih, (dih.shape[0], H))
    dih_ref[...] = dih_b

    colsum_d = jnp.sum(wd_ref[...], axis=0, keepdims=True)
    dmod = jax.nn.sigmoid(dih * colsum_d + bd_ref[...])
    f = f_ref[...]
    fout_ref[...] = f + (jnp.dot(f, we_ref[...], preferred_element_type=jnp.float32)
                         + be_ref[...]) * dmod


def _node_body(h_ref, du_ref, ws_ref, bs_ref, wa_ref, ba_ref,
               hout_ref, ang_ref):
    du = du_ref[...]
    ang = du[:, 0:1] ** 2 + du[:, 1:2] ** 2 + du[:, 2:3] ** 2
    ang_ref[...] = jnp.broadcast_to(ang, (ang.shape[0], H))
    colsum_a = jnp.sum(wa_ref[...], axis=0, keepdims=True)
    amod = jax.nn.sigmoid(ang * colsum_a + ba_ref[...])
    h = h_ref[...]
    hout_ref[...] = h + (jnp.dot(h, ws_ref[...], preferred_element_type=jnp.float32)
                         + bs_ref[...]) * amod


def kernel(h, v, f, pos, edge_index, edge_rbf,
           lin_msg_w, lin_msg_b, lin_vec_w, lin_vec_b,
           lin_scalar_w, lin_scalar_b, lin_edge_w, lin_edge_b,
           lin_angular_w, lin_angular_b, lin_dihedral_w, lin_dihedral_b):
    n = pos.shape[0]
    e = edge_index.shape[1]
    row = edge_index[0]
    col = edge_index[1]

    ev3 = pos[col] - pos[row]
    ev = jnp.pad(ev3, ((0, 0), (0, 5)))
    geom = pl.pallas_call(
        _geom_body,
        grid=(e // EB,),
        in_specs=[pl.BlockSpec((EB, 8), lambda i: (i, 0))],
        out_specs=pl.BlockSpec((EB, 8), lambda i: (i, 0)),
        out_shape=jax.ShapeDtypeStruct((e, 8), jnp.float32),
    )(ev)
    u = geom[:, 0:3]

    du = jnp.zeros((n, 3), jnp.float32).at[row].add(u).at[col].add(-u)
    du_row = du[row]
    du_col = du[col]
    du2 = jnp.concatenate([du_row, jnp.zeros((e, 1), jnp.float32),
                           du_col, jnp.zeros((e, 1), jnp.float32)], axis=1)

    hc = h[col]
    hr = h[row]
    vr = v[row].reshape(e, 3 * H)
    rbf_p = jnp.pad(edge_rbf, ((0, 0), (0, RP - R)))

    w1a = lin_msg_w[:H]
    w1b = lin_msg_w[H:2 * H]
    w1c = jnp.pad(lin_msg_w[2 * H:], ((0, RP - R), (0, 0)))
    b1 = lin_msg_b.reshape(1, H)
    bv = lin_vec_b.reshape(1, 2 * H)
    be = lin_edge_b.reshape(1, H)
    bd = lin_dihedral_b.reshape(1, H)
    bs = lin_scalar_b.reshape(1, H)
    ba = lin_angular_b.reshape(1, H)

    wspec = pl.BlockSpec(None, lambda i: (0, 0))
    espec = lambda w: pl.BlockSpec((EB, w), lambda i: (i, 0))
    msg, f_updated, dihedral_info = pl.pallas_call(
        _edge_body,
        grid=(e // EB,),
        in_specs=[espec(H), espec(H), espec(RP), espec(3 * H), espec(H),
                  espec(8), espec(8),
                  wspec, wspec, wspec, wspec, wspec, wspec,
                  wspec, wspec, wspec, wspec],
        out_specs=[espec(3 * H), espec(H), espec(H)],
        out_shape=[jax.ShapeDtypeStruct((e, 3 * H), jnp.float32),
                   jax.ShapeDtypeStruct((e, H), jnp.float32),
                   jax.ShapeDtypeStruct((e, H), jnp.float32)],
    )(hc, hr, rbf_p, vr, f, geom, du2,
      w1a, w1b, w1c, b1, lin_vec_w, bv,
      lin_edge_w, be, lin_dihedral_w, bd)

    v_upd = jnp.zeros((n, 3 * H), jnp.float32).at[col].add(msg)
    v_updated = v + v_upd.reshape(n, 3, H)

    du_p = jnp.pad(du, ((0, 0), (0, 5)))
    h_updated, angular_info = pl.pallas_call(
        _node_body,
        grid=(n // NB,),
        in_specs=[pl.BlockSpec((NB, H), lambda i: (i, 0)),
                  pl.BlockSpec((NB, 8), lambda i: (i, 0)),
                  wspec, wspec, wspec, wspec],
        out_specs=[pl.BlockSpec((NB, H), lambda i: (i, 0)),
                   pl.BlockSpec((NB, H), lambda i: (i, 0))],
        out_shape=[jax.ShapeDtypeStruct((n, H), jnp.float32),
                   jax.ShapeDtypeStruct((n, H), jnp.float32)],
    )(h, du_p, lin_scalar_w, bs, lin_angular_w, ba)

    return (h_updated, v_updated, f_updated, angular_info, dihedral_info, du)


# speedup vs baseline: 7.4728x; 2.2040x over previous
import functools

import jax
import jax.numpy as jnp
from jax import lax
from jax.experimental import pallas as pl
from jax.experimental.pallas import tpu as pltpu
from jax.experimental.pallas import tpu_sc as plsc

H = 256
R = 50
RP = 64
CUTOFF = 10.0

EB = 640
NB = 1000

NSLICE = 6
SW = 128
TECS = 16
KB = 80


def _sc_pos_gather_call(pos8, rowi, coli):
    e = rowi.shape[0]
    NW = 2 * TECS
    per_w = e // NW
    K = 200
    nb = per_w // K
    mesh = plsc.VectorSubcoreMesh(core_axis_name="c", subcore_axis_name="s")

    @functools.partial(
        pl.kernel, mesh=mesh,
        out_type=[jax.ShapeDtypeStruct((e, 128), jnp.float32),
                  jax.ShapeDtypeStruct((e, 128), jnp.float32)],
        scratch_types=[
            pltpu.VMEM((K,), jnp.int32),
            pltpu.VMEM((K, 128), jnp.float32),
            pltpu.SemaphoreType.DMA,
        ])
    def body(pos_hbm, row_hbm, col_hbm, pr_hbm, pc_hbm, idx_v, st_v, sem):
        c = lax.axis_index("c")
        t = lax.axis_index("s")
        wbase = (t * 2 + c) * per_w

        def batch(b, _):
            base = wbase + b * K
            pltpu.sync_copy(row_hbm.at[pl.ds(base, K)], idx_v)
            pltpu.async_copy(pos_hbm.at[idx_v], st_v, sem).wait()
            pltpu.sync_copy(st_v, pr_hbm.at[pl.ds(base, K)])
            pltpu.sync_copy(col_hbm.at[pl.ds(base, K)], idx_v)
            pltpu.async_copy(pos_hbm.at[idx_v], st_v, sem).wait()
            pltpu.sync_copy(st_v, pc_hbm.at[pl.ds(base, K)])
            return 0

        lax.fori_loop(0, nb, batch, 0)

    return body(pos8, rowi, coli)


def _sc_hv_gather_call(h, v2, rowi, coli):
    e = rowi.shape[0]
    NW = 2 * TECS
    per_w = e // NW
    K1 = 200
    K2 = 40
    nb1 = per_w // K1
    nb2 = per_w // K2
    mesh = plsc.VectorSubcoreMesh(core_axis_name="c", subcore_axis_name="s")

    @functools.partial(
        pl.kernel, mesh=mesh,
        out_type=[jax.ShapeDtypeStruct((e, H), jnp.float32),
                  jax.ShapeDtypeStruct((e, H), jnp.float32),
                  jax.ShapeDtypeStruct((e, 3 * H), jnp.float32)],
        scratch_types=[
            pltpu.VMEM((K1,), jnp.int32),
            pltpu.VMEM((K2,), jnp.int32),
            pltpu.VMEM((K1, H), jnp.float32),
            pltpu.VMEM((K2, 3 * H), jnp.float32),
            pltpu.SemaphoreType.DMA,
        ])
    def body(h_hbm, v_hbm, row_hbm, col_hbm, hc_hbm, hr_hbm, vr_hbm,
             idx1_v, idx2_v, sh_v, sv_v, sem):
        c = lax.axis_index("c")
        t = lax.axis_index("s")
        wbase = (t * 2 + c) * per_w

        def hbatch(b, _):
            base = wbase + b * K1
            pltpu.sync_copy(col_hbm.at[pl.ds(base, K1)], idx1_v)
            pltpu.async_copy(h_hbm.at[idx1_v], sh_v, sem).wait()
            pltpu.sync_copy(sh_v, hc_hbm.at[pl.ds(base, K1)])
            pltpu.sync_copy(row_hbm.at[pl.ds(base, K1)], idx1_v)
            pltpu.async_copy(h_hbm.at[idx1_v], sh_v, sem).wait()
            pltpu.sync_copy(sh_v, hr_hbm.at[pl.ds(base, K1)])
            return 0

        def vbatch(b, _):
            base = wbase + b * K2
            pltpu.sync_copy(row_hbm.at[pl.ds(base, K2)], idx2_v)
            pltpu.async_copy(v_hbm.at[idx2_v], sv_v, sem).wait()
            pltpu.sync_copy(sv_v, vr_hbm.at[pl.ds(base, K2)])
            return 0

        lax.fori_loop(0, nb1, hbatch, 0)
        lax.fori_loop(0, nb2, vbatch, 0)

    return body(h, v2, rowi, coli)


def _sc_du_scatter_call(pu, nu, rowi, coli, zeros_n8, n):
    e = rowi.shape[0]
    per_tec = e // TECS
    K = 200
    nb = per_tec // K
    RC = 80
    n_chunks = n // RC
    k_rounds = (n_chunks + TECS - 1) // TECS
    mesh = plsc.VectorSubcoreMesh(core_axis_name="c", subcore_axis_name="s")

    @functools.partial(
        pl.kernel, mesh=mesh,
        out_type=jax.ShapeDtypeStruct((n, 128), jnp.float32),
        scratch_types=[
            pltpu.VMEM((K,), jnp.int32),
            pltpu.VMEM((K, 128), jnp.float32),
            pltpu.VMEM_SHARED((n, 128), jnp.float32),
        ])
    def body(pu_hbm, nu_hbm, row_hbm, col_hbm, zero_hbm, out_hbm,
             idx_v, stage_v, acc_sh):
        c = lax.axis_index("c")
        t = lax.axis_index("s")

        @pl.when(c == 0)
        def _():
            for k in range(k_rounds):
                cid = k * TECS + t

                @pl.when(cid < n_chunks)
                def _():
                    pltpu.sync_copy(zero_hbm.at[pl.ds(cid * RC, RC)],
                                    acc_sh.at[pl.ds(cid * RC, RC)])
            plsc.subcore_barrier()
            ebase = t * per_tec

            def batch(b, _):
                base = ebase + b * K
                pltpu.sync_copy(row_hbm.at[pl.ds(base, K)], idx_v)
                pltpu.sync_copy(pu_hbm.at[pl.ds(base, K)], stage_v)
                pltpu.sync_copy(stage_v, acc_sh.at[idx_v], add=True)
                pltpu.sync_copy(col_hbm.at[pl.ds(base, K)], idx_v)
                pltpu.sync_copy(nu_hbm.at[pl.ds(base, K)], stage_v)
                pltpu.sync_copy(stage_v, acc_sh.at[idx_v], add=True)
                return 0

            lax.fori_loop(0, nb, batch, 0)
            plsc.subcore_barrier()
            for k in range(k_rounds):
                cid = k * TECS + t

                @pl.when(cid < n_chunks)
                def _():
                    pltpu.sync_copy(acc_sh.at[pl.ds(cid * RC, RC)],
                                    out_hbm.at[pl.ds(cid * RC, RC)])

    return body(pu, nu, rowi, coli, zeros_n8)


def _sc_du_gather_call(du8, rowi, coli):
    e = rowi.shape[0]
    NW = 2 * TECS
    per_w = e // NW
    K = 200
    nb = per_w // K
    mesh = plsc.VectorSubcoreMesh(core_axis_name="c", subcore_axis_name="s")

    @functools.partial(
        pl.kernel, mesh=mesh,
        out_type=[jax.ShapeDtypeStruct((e, 128), jnp.float32),
                  jax.ShapeDtypeStruct((e, 128), jnp.float32)],
        scratch_types=[
            pltpu.VMEM((K,), jnp.int32),
            pltpu.VMEM((K, 128), jnp.float32),
            pltpu.SemaphoreType.DMA,
        ])
    def body(du_hbm, row_hbm, col_hbm, dur_hbm, duc_hbm, idx_v, st_v, sem):
        c = lax.axis_index("c")
        t = lax.axis_index("s")
        wbase = (t * 2 + c) * per_w

        def batch(b, _):
            base = wbase + b * K
            pltpu.sync_copy(row_hbm.at[pl.ds(base, K)], idx_v)
            pltpu.async_copy(du_hbm.at[idx_v], st_v, sem).wait()
            pltpu.sync_copy(st_v, dur_hbm.at[pl.ds(base, K)])
            pltpu.sync_copy(col_hbm.at[pl.ds(base, K)], idx_v)
            pltpu.async_copy(du_hbm.at[idx_v], st_v, sem).wait()
            pltpu.sync_copy(st_v, duc_hbm.at[pl.ds(base, K)])
            return 0

        lax.fori_loop(0, nb, batch, 0)

    return body(du8, rowi, coli)


def _sc_scatter_call(msg6, col, zeros_nw, n):
    e = col.shape[0]
    per_tec = e // TECS
    nb = per_tec // KB
    RC = 80
    n_chunks = n // RC
    k_rounds = (n_chunks + TECS - 1) // TECS
    mesh = plsc.VectorSubcoreMesh(core_axis_name="c", subcore_axis_name="s")

    @functools.partial(
        pl.kernel, mesh=mesh,
        out_type=jax.ShapeDtypeStruct((NSLICE, n, SW), jnp.float32),
        scratch_types=[
            pltpu.VMEM((KB,), jnp.int32),
            pltpu.VMEM((KB, SW), jnp.float32),
            pltpu.VMEM_SHARED((n, SW), jnp.float32),
        ])
    def body(msg_hbm, col_hbm, zero_hbm, out_hbm, idx_v, stage_v, acc_sh):
        c = lax.axis_index("c")
        t = lax.axis_index("s")
        ebase = t * per_tec
        for s_local in range(NSLICE // 2):
            s = c * (NSLICE // 2) + s_local
            for k in range(k_rounds):
                cid = k * TECS + t

                @pl.when(cid < n_chunks)
                def _():
                    pltpu.sync_copy(zero_hbm.at[pl.ds(cid * RC, RC)],
                                    acc_sh.at[pl.ds(cid * RC, RC)])
            plsc.subcore_barrier()

            def batch(b, _):
                base = ebase + b * KB
                pltpu.sync_copy(col_hbm.at[pl.ds(base, KB)], idx_v)
                pltpu.sync_copy(msg_hbm.at[s, pl.ds(base, KB)], stage_v)
                pltpu.sync_copy(stage_v, acc_sh.at[idx_v], add=True)
                return 0

            lax.fori_loop(0, nb, batch, 0)
            plsc.subcore_barrier()
            for k in range(k_rounds):
                cid = k * TECS + t

                @pl.when(cid < n_chunks)
                def _():
                    pltpu.sync_copy(acc_sh.at[pl.ds(cid * RC, RC)],
                                    out_hbm.at[s, pl.ds(cid * RC, RC)])
            plsc.subcore_barrier()

    return body(msg6, col, zeros_nw)


def _geom_body(pc_ref, pr_ref, geom_ref, negu_ref):
    pc = pc_ref[...]
    pr = pr_ref[...]
    ev = pc - pr
    e0 = ev[:, 0:1]
    e1 = ev[:, 1:2]
    e2 = ev[:, 2:3]
    dist = jnp.sqrt(e0 * e0 + e1 * e1 + e2 * e2) + 1e-8
    inv = 1.0 / dist
    u0 = e0 * inv
    u1 = e1 * inv
    u2 = e2 * inv
    cw = 0.5 * (jnp.cos(jnp.pi * dist / CUTOFF) + 1.0)
    cw = jnp.where(dist < CUTOFF, cw, 0.0)
    zero = jnp.zeros((u0.shape[0], 124), jnp.float32)
    geom_ref[...] = jnp.concatenate([u0, u1, u2, cw, zero], axis=1)
    negu_ref[...] = jnp.concatenate(
        [-u0, -u1, -u2, jnp.zeros_like(u0), zero], axis=1)


def _edge_body(hc_ref, hr_ref, rbf_ref, vr_ref, f_ref, geom_ref,
               dur_ref, duc_ref,
               w1a_ref, w1b_ref, w1c_ref, b1_ref, wv_ref, bv_ref,
               we_ref, be_ref, wd_ref, bd_ref,
               msg_ref, fout_ref, dih_ref):
    hc = hc_ref[...]
    hr = hr_ref[...]
    rbf = rbf_ref[...]
    geom = geom_ref[...]
    u0 = geom[:, 0:1]
    u1 = geom[:, 1:2]
    u2 = geom[:, 2:3]
    cw = geom[:, 3:4]

    sm = (jnp.dot(hc, w1a_ref[...], preferred_element_type=jnp.float32)
          + jnp.dot(hr, w1b_ref[...], preferred_element_type=jnp.float32)
          + jnp.dot(rbf, w1c_ref[...], preferred_element_type=jnp.float32)
          + b1_ref[...])
    vw = jnp.dot(sm, wv_ref[...], preferred_element_type=jnp.float32) + bv_ref[...]
    w1 = vw[:, :H] * cw
    w2 = vw[:, H:] * cw
    vr = vr_ref[...]
    m0 = w1 * u0 + w2 * vr[:, 0:H]
    m1 = w1 * u1 + w2 * vr[:, H:2 * H]
    m2 = w1 * u2 + w2 * vr[:, 2 * H:3 * H]
    msg = jnp.concatenate([m0, m1, m2], axis=1)
    for s in range(NSLICE):
        msg_ref[s] = msg[:, SW * s:SW * (s + 1)]

    dur = dur_ref[...]
    duc = duc_ref[...]
    a0 = dur[:, 0:1]
    a1 = dur[:, 1:2]
    a2 = dur[:, 2:3]
    b0 = duc[:, 0:1]
    b1 = duc[:, 1:2]
    b2 = duc[:, 2:3]
    dvi = a0 * u0 + a1 * u1 + a2 * u2
    dvj = -(b0 * u0 + b1 * u1 + b2 * u2)
    w_ij0 = a0 - dvi * u0
    w_ij1 = a1 - dvi * u1
    w_ij2 = a2 - dvi * u2
    w_ji0 = b0 + dvj * u0
    w_ji1 = b1 + dvj * u1
    w_ji2 = b2 + dvj * u2
    dih = w_ij0 * w_ji0 + w_ij1 * w_ji1 + w_ij2 * w_ji2
    dih_ref[...] = jnp.broadcast_to(dih, (dih.shape[0], H))

    wd_bf = wd_ref[...].astype(jnp.bfloat16).astype(jnp.float32)
    colsum_d = jnp.sum(wd_bf, axis=0, keepdims=True)
    dih_bf = dih.astype(jnp.bfloat16).astype(jnp.float32)
    dmod = jax.nn.sigmoid(dih_bf * colsum_d + bd_ref[...])
    f = f_ref[...]
    fout_ref[...] = f + (jnp.dot(f, we_ref[...], preferred_element_type=jnp.float32)
                         + be_ref[...]) * dmod


def _node_body(h_ref, du_ref, v_ref, vu_ref, ws_ref, bs_ref, wa_ref, ba_ref,
               hout_ref, ang_ref, vout_ref):
    du = du_ref[...]
    ang = du[:, 0:1] ** 2 + du[:, 1:2] ** 2 + du[:, 2:3] ** 2
    ang_ref[...] = jnp.broadcast_to(ang, (ang.shape[0], H))
    wa_bf = wa_ref[...].astype(jnp.bfloat16).astype(jnp.float32)
    colsum_a = jnp.sum(wa_bf, axis=0, keepdims=True)
    ang_bf = ang.astype(jnp.bfloat16).astype(jnp.float32)
    amod = jax.nn.sigmoid(ang_bf * colsum_a + ba_ref[...])
    h = h_ref[...]
    hout_ref[...] = h + (jnp.dot(h, ws_ref[...], preferred_element_type=jnp.float32)
                         + bs_ref[...]) * amod
    v = v_ref[...]
    for s in range(NSLICE):
        vout_ref[:, SW * s:SW * (s + 1)] = v[:, SW * s:SW * (s + 1)] + vu_ref[s]


def kernel(h, v, f, pos, edge_index, edge_rbf,
           lin_msg_w, lin_msg_b, lin_vec_w, lin_vec_b,
           lin_scalar_w, lin_scalar_b, lin_edge_w, lin_edge_b,
           lin_angular_w, lin_angular_b, lin_dihedral_w, lin_dihedral_b):
    n = pos.shape[0]
    e = edge_index.shape[1]
    row = edge_index[0]
    col = edge_index[1]

    pos8 = jnp.pad(pos, ((0, 0), (0, 125)))
    pr8, pc8 = _sc_pos_gather_call(pos8, row, col)
    v2 = v.reshape(n, 3 * H)
    hc, hr, vr = _sc_hv_gather_call(h, v2, row, col)

    geom, negu = pl.pallas_call(
        _geom_body,
        grid=(e // EB,),
        in_specs=[pl.BlockSpec((EB, 128), lambda i: (i, 0)),
                  pl.BlockSpec((EB, 128), lambda i: (i, 0))],
        out_specs=[pl.BlockSpec((EB, 128), lambda i: (i, 0)),
                   pl.BlockSpec((EB, 128), lambda i: (i, 0))],
        out_shape=[jax.ShapeDtypeStruct((e, 128), jnp.float32),
                   jax.ShapeDtypeStruct((e, 128), jnp.float32)],
    )(pc8, pr8)

    zeros_n8 = jnp.zeros((n, 128), jnp.float32)
    du8 = _sc_du_scatter_call(geom, negu, row, col, zeros_n8, n)
    dur8, duc8 = _sc_du_gather_call(du8, row, col)
    du = du8[:, :3]

    rbf_p = jnp.pad(edge_rbf, ((0, 0), (0, RP - R)))
    w1a = lin_msg_w[:H]
    w1b = lin_msg_w[H:2 * H]
    w1c = jnp.pad(lin_msg_w[2 * H:], ((0, RP - R), (0, 0)))
    b1 = lin_msg_b.reshape(1, H)
    bv = lin_vec_b.reshape(1, 2 * H)
    be = lin_edge_b.reshape(1, H)
    bd = lin_dihedral_b.reshape(1, H)
    bs = lin_scalar_b.reshape(1, H)
    ba = lin_angular_b.reshape(1, H)

    wspec = pl.BlockSpec(None, lambda i: (0, 0))
    espec = lambda w: pl.BlockSpec((EB, w), lambda i: (i, 0))
    msg6, f_updated, dihedral_info = pl.pallas_call(
        _edge_body,
        grid=(e // EB,),
        in_specs=[espec(H), espec(H), espec(RP), espec(3 * H), espec(H),
                  espec(128), espec(128), espec(128),
                  wspec, wspec, wspec, wspec, wspec, wspec,
                  wspec, wspec, wspec, wspec],
        out_specs=[pl.BlockSpec((NSLICE, EB, SW), lambda i: (0, i, 0)),
                   espec(H), espec(H)],
        out_shape=[jax.ShapeDtypeStruct((NSLICE, e, SW), jnp.float32),
                   jax.ShapeDtypeStruct((e, H), jnp.float32),
                   jax.ShapeDtypeStruct((e, H), jnp.float32)],
    )(hc, hr, rbf_p, vr, f, geom, dur8, duc8,
      w1a, w1b, w1c, b1, lin_vec_w, bv,
      lin_edge_w, be, lin_dihedral_w, bd)

    zeros_nw = jnp.zeros((n, SW), jnp.float32)
    vupd6 = _sc_scatter_call(msg6, col, zeros_nw, n)

    du_p = du8
    h_updated, angular_info, v_updated = pl.pallas_call(
        _node_body,
        grid=(n // NB,),
        in_specs=[pl.BlockSpec((NB, H), lambda i: (i, 0)),
                  pl.BlockSpec((NB, 128), lambda i: (i, 0)),
                  pl.BlockSpec((NB, 3 * H), lambda i: (i, 0)),
                  pl.BlockSpec((NSLICE, NB, SW), lambda i: (0, i, 0)),
                  wspec, wspec, wspec, wspec],
        out_specs=[pl.BlockSpec((NB, H), lambda i: (i, 0)),
                   pl.BlockSpec((NB, H), lambda i: (i, 0)),
                   pl.BlockSpec((NB, 3 * H), lambda i: (i, 0))],
        out_shape=[jax.ShapeDtypeStruct((n, H), jnp.float32),
                   jax.ShapeDtypeStruct((n, H), jnp.float32),
                   jax.ShapeDtypeStruct((n, 3 * H), jnp.float32)],
    )(h, du_p, v2, vupd6, lin_scalar_w, bs, lin_angular_w, ba)
    v_updated = v_updated.reshape(n, 3, H)

    return (h_updated, v_updated, f_updated, angular_info, dihedral_info, du)


# speedup vs baseline: 8.7691x; 1.1735x over previous
import functools

import jax
import jax.numpy as jnp
from jax import lax
from jax.experimental import pallas as pl
from jax.experimental.pallas import tpu as pltpu
from jax.experimental.pallas import tpu_sc as plsc

H = 256
R = 50
RP = 64
CUTOFF = 10.0

EB = 1000
NB = 1000

NSLICE = 6
SW = 128
TECS = 16
KB = 80


def _sc_pos_gather_call(pos8, rowi, coli):
    e = rowi.shape[0]
    NW = 2 * TECS
    per_w = e // NW
    K = 200
    nb = per_w // K
    mesh = plsc.VectorSubcoreMesh(core_axis_name="c", subcore_axis_name="s")

    @functools.partial(
        pl.kernel, mesh=mesh,
        out_type=[jax.ShapeDtypeStruct((e, 128), jnp.float32),
                  jax.ShapeDtypeStruct((e, 128), jnp.float32)],
        scratch_types=[
            pltpu.VMEM((K,), jnp.int32),
            pltpu.VMEM((K, 128), jnp.float32),
            pltpu.SemaphoreType.DMA,
        ])
    def body(pos_hbm, row_hbm, col_hbm, pr_hbm, pc_hbm, idx_v, st_v, sem):
        c = lax.axis_index("c")
        t = lax.axis_index("s")
        wbase = (t * 2 + c) * per_w

        def batch(b, _):
            base = wbase + b * K
            pltpu.sync_copy(row_hbm.at[pl.ds(base, K)], idx_v)
            pltpu.async_copy(pos_hbm.at[idx_v], st_v, sem).wait()
            pltpu.sync_copy(st_v, pr_hbm.at[pl.ds(base, K)])
            pltpu.sync_copy(col_hbm.at[pl.ds(base, K)], idx_v)
            pltpu.async_copy(pos_hbm.at[idx_v], st_v, sem).wait()
            pltpu.sync_copy(st_v, pc_hbm.at[pl.ds(base, K)])
            return 0

        lax.fori_loop(0, nb, batch, 0)

    return body(pos8, rowi, coli)


def _sc_hv_gather_call(h, v2, rowi, coli):
    e = rowi.shape[0]
    NW = 2 * TECS
    KH = 80
    nch = e // KH
    nrounds = (nch + NW - 1) // NW
    mesh = plsc.VectorSubcoreMesh(core_axis_name="c", subcore_axis_name="s")

    @functools.partial(
        pl.kernel, mesh=mesh,
        out_type=[jax.ShapeDtypeStruct((e, H), jnp.float32),
                  jax.ShapeDtypeStruct((e, H), jnp.float32),
                  jax.ShapeDtypeStruct((e, 3 * H), jnp.float32)],
        scratch_types=[
            pltpu.VMEM((KH,), jnp.int32),
            pltpu.VMEM((KH, H), jnp.float32),
            pltpu.VMEM((KH, 3 * H), jnp.float32),
            pltpu.SemaphoreType.DMA,
        ])
    def body(h_hbm, v_hbm, row_hbm, col_hbm, hc_hbm, hr_hbm, vr_hbm,
             idx_v, sh_v, sv_v, sem):
        c = lax.axis_index("c")
        t = lax.axis_index("s")
        wid = t * 2 + c

        def batch(b, _):
            cid = b * NW + wid

            @pl.when(cid < nch)
            def _():
                base = cid * KH
                pltpu.sync_copy(col_hbm.at[pl.ds(base, KH)], idx_v)
                pltpu.async_copy(h_hbm.at[idx_v], sh_v, sem).wait()
                pltpu.sync_copy(sh_v, hc_hbm.at[pl.ds(base, KH)])
                pltpu.sync_copy(row_hbm.at[pl.ds(base, KH)], idx_v)
                pltpu.async_copy(h_hbm.at[idx_v], sh_v, sem).wait()
                pltpu.sync_copy(sh_v, hr_hbm.at[pl.ds(base, KH)])
                pltpu.async_copy(v_hbm.at[idx_v], sv_v, sem).wait()
                pltpu.sync_copy(sv_v, vr_hbm.at[pl.ds(base, KH)])
            return 0

        lax.fori_loop(0, nrounds, batch, 0)

    return body(h, v2, rowi, coli)


def _sc_du_scatter_call(pu, nu, rowi, coli, zeros_n8, n):
    e = rowi.shape[0]
    per_tec = e // TECS
    K = 200
    nb = per_tec // K
    RC = 80
    n_chunks = n // RC
    k_rounds = (n_chunks + TECS - 1) // TECS
    mesh = plsc.VectorSubcoreMesh(core_axis_name="c", subcore_axis_name="s")

    @functools.partial(
        pl.kernel, mesh=mesh,
        out_type=jax.ShapeDtypeStruct((n, 128), jnp.float32),
        scratch_types=[
            pltpu.VMEM((K,), jnp.int32),
            pltpu.VMEM((K, 128), jnp.float32),
            pltpu.VMEM_SHARED((n, 128), jnp.float32),
        ])
    def body(pu_hbm, nu_hbm, row_hbm, col_hbm, zero_hbm, out_hbm,
             idx_v, stage_v, acc_sh):
        c = lax.axis_index("c")
        t = lax.axis_index("s")

        @pl.when(c == 0)
        def _():
            for k in range(k_rounds):
                cid = k * TECS + t

                @pl.when(cid < n_chunks)
                def _():
                    pltpu.sync_copy(zero_hbm.at[pl.ds(cid * RC, RC)],
                                    acc_sh.at[pl.ds(cid * RC, RC)])
            plsc.subcore_barrier()
            ebase = t * per_tec

            def batch(b, _):
                base = ebase + b * K
                pltpu.sync_copy(row_hbm.at[pl.ds(base, K)], idx_v)
                pltpu.sync_copy(pu_hbm.at[pl.ds(base, K)], stage_v)
                pltpu.sync_copy(stage_v, acc_sh.at[idx_v], add=True)
                pltpu.sync_copy(col_hbm.at[pl.ds(base, K)], idx_v)
                pltpu.sync_copy(nu_hbm.at[pl.ds(base, K)], stage_v)
                pltpu.sync_copy(stage_v, acc_sh.at[idx_v], add=True)
                return 0

            lax.fori_loop(0, nb, batch, 0)
            plsc.subcore_barrier()
            for k in range(k_rounds):
                cid = k * TECS + t

                @pl.when(cid < n_chunks)
                def _():
                    pltpu.sync_copy(acc_sh.at[pl.ds(cid * RC, RC)],
                                    out_hbm.at[pl.ds(cid * RC, RC)])

    return body(pu, nu, rowi, coli, zeros_n8)


def _sc_du_gather_call(du8, rowi, coli):
    e = rowi.shape[0]
    NW = 2 * TECS
    per_w = e // NW
    K = 200
    nb = per_w // K
    mesh = plsc.VectorSubcoreMesh(core_axis_name="c", subcore_axis_name="s")

    @functools.partial(
        pl.kernel, mesh=mesh,
        out_type=[jax.ShapeDtypeStruct((e, 128), jnp.float32),
                  jax.ShapeDtypeStruct((e, 128), jnp.float32)],
        scratch_types=[
            pltpu.VMEM((K,), jnp.int32),
            pltpu.VMEM((K, 128), jnp.float32),
            pltpu.SemaphoreType.DMA,
        ])
    def body(du_hbm, row_hbm, col_hbm, dur_hbm, duc_hbm, idx_v, st_v, sem):
        c = lax.axis_index("c")
        t = lax.axis_index("s")
        wbase = (t * 2 + c) * per_w

        def batch(b, _):
            base = wbase + b * K
            pltpu.sync_copy(row_hbm.at[pl.ds(base, K)], idx_v)
            pltpu.async_copy(du_hbm.at[idx_v], st_v, sem).wait()
            pltpu.sync_copy(st_v, dur_hbm.at[pl.ds(base, K)])
            pltpu.sync_copy(col_hbm.at[pl.ds(base, K)], idx_v)
            pltpu.async_copy(du_hbm.at[idx_v], st_v, sem).wait()
            pltpu.sync_copy(st_v, duc_hbm.at[pl.ds(base, K)])
            return 0

        lax.fori_loop(0, nb, batch, 0)

    return body(du8, rowi, coli)


def _sc_scatter_call(msg6, col, zeros_nw, n):
    e = col.shape[0]
    per_tec = e // TECS
    nb = per_tec // KB
    RC = 80
    n_chunks = n // RC
    k_rounds = (n_chunks + TECS - 1) // TECS
    mesh = plsc.VectorSubcoreMesh(core_axis_name="c", subcore_axis_name="s")

    @functools.partial(
        pl.kernel, mesh=mesh,
        out_type=jax.ShapeDtypeStruct((NSLICE, n, SW), jnp.float32),
        scratch_types=[
            pltpu.VMEM((KB,), jnp.int32),
            pltpu.VMEM((KB,), jnp.int32),
            pltpu.VMEM((KB, SW), jnp.float32),
            pltpu.VMEM((KB, SW), jnp.float32),
            pltpu.VMEM_SHARED((n, SW), jnp.float32),
            pltpu.SemaphoreType.DMA,
            pltpu.SemaphoreType.DMA,
            pltpu.SemaphoreType.DMA,
            pltpu.SemaphoreType.DMA,
        ])
    def body(msg_hbm, col_hbm, zero_hbm, out_hbm,
             idx0_v, idx1_v, st0_v, st1_v, acc_sh,
             semi0, semi1, semm0, semm1):
        c = lax.axis_index("c")
        t = lax.axis_index("s")
        ebase = t * per_tec
        idx_bufs = (idx0_v, idx1_v)
        st_bufs = (st0_v, st1_v)
        sems_i = (semi0, semi1)
        sems_m = (semm0, semm1)

        def load(b, j):
            base = ebase + b * KB
            pltpu.async_copy(col_hbm.at[pl.ds(base, KB)], idx_bufs[j],
                             sems_i[j])
            pltpu.async_copy(msg_hbm.at[s, pl.ds(base, KB)], st_bufs[j],
                             sems_m[j])

        def drain_scatter(b, j):
            base = ebase + b * KB
            pltpu.make_async_copy(col_hbm.at[pl.ds(base, KB)],
                                  idx_bufs[j], sems_i[j]).wait()
            pltpu.make_async_copy(msg_hbm.at[s, pl.ds(base, KB)],
                                  st_bufs[j], sems_m[j]).wait()
            pltpu.sync_copy(st_bufs[j], acc_sh.at[idx_bufs[j]], add=True)

        for s_local in range(NSLICE // 2):
            s = c * (NSLICE // 2) + s_local
            for k in range(k_rounds):
                cid = k * TECS + t

                @pl.when(cid < n_chunks)
                def _():
                    pltpu.sync_copy(zero_hbm.at[pl.ds(cid * RC, RC)],
                                    acc_sh.at[pl.ds(cid * RC, RC)])
            plsc.subcore_barrier()

            load(0, 0)

            def pair(k2, _):
                b0 = 2 * k2
                load(b0 + 1, 1)
                drain_scatter(b0, 0)

                @pl.when(b0 + 2 < nb)
                def _():
                    load(b0 + 2, 0)
                drain_scatter(b0 + 1, 1)
                return 0

            lax.fori_loop(0, nb // 2, pair, 0)
            if nb % 2 == 1:
                drain_scatter(nb - 1, 0)
            plsc.subcore_barrier()
            for k in range(k_rounds):
                cid = k * TECS + t

                @pl.when(cid < n_chunks)
                def _():
                    pltpu.sync_copy(acc_sh.at[pl.ds(cid * RC, RC)],
                                    out_hbm.at[s, pl.ds(cid * RC, RC)])
            plsc.subcore_barrier()

    return body(msg6, col, zeros_nw)


def _geom_body(pc_ref, pr_ref, geom_ref, negu_ref):
    pc = pc_ref[...]
    pr = pr_ref[...]
    ev = pc - pr
    e0 = ev[:, 0:1]
    e1 = ev[:, 1:2]
    e2 = ev[:, 2:3]
    dist = jnp.sqrt(e0 * e0 + e1 * e1 + e2 * e2) + 1e-8
    inv = 1.0 / dist
    u0 = e0 * inv
    u1 = e1 * inv
    u2 = e2 * inv
    cw = 0.5 * (jnp.cos(jnp.pi * dist / CUTOFF) + 1.0)
    cw = jnp.where(dist < CUTOFF, cw, 0.0)
    zero = jnp.zeros((u0.shape[0], 124), jnp.float32)
    geom_ref[...] = jnp.concatenate([u0, u1, u2, cw, zero], axis=1)
    negu_ref[...] = jnp.concatenate(
        [-u0, -u1, -u2, jnp.zeros_like(u0), zero], axis=1)


def _edge_body(hc_ref, hr_ref, rbf_ref, vr_ref, f_ref, geom_ref,
               dur_ref, duc_ref,
               w1a_ref, w1b_ref, w1c_ref, b1_ref, wv_ref, bv_ref,
               we_ref, be_ref, wd_ref, bd_ref,
               msg_ref, fout_ref, dih_ref):
    hc = hc_ref[...]
    hr = hr_ref[...]
    rbf = rbf_ref[...]
    geom = geom_ref[...]
    u0 = geom[:, 0:1]
    u1 = geom[:, 1:2]
    u2 = geom[:, 2:3]
    cw = geom[:, 3:4]

    sm = (jnp.dot(hc, w1a_ref[...], preferred_element_type=jnp.float32)
          + jnp.dot(hr, w1b_ref[...], preferred_element_type=jnp.float32)
          + jnp.dot(rbf, w1c_ref[...], preferred_element_type=jnp.float32)
          + b1_ref[...])
    vw = jnp.dot(sm, wv_ref[...], preferred_element_type=jnp.float32) + bv_ref[...]
    w1 = vw[:, :H] * cw
    w2 = vw[:, H:] * cw
    vr = vr_ref[...]
    us = (u0, u1, u2)
    for s in range(NSLICE):
        d, half = s // 2, s % 2
        msg_ref[s] = (w1[:, SW * half:SW * (half + 1)] * us[d]
                      + w2[:, SW * half:SW * (half + 1)]
                      * vr[:, H * d + SW * half:H * d + SW * (half + 1)])

    dur = dur_ref[...]
    duc = duc_ref[...]
    a0 = dur[:, 0:1]
    a1 = dur[:, 1:2]
    a2 = dur[:, 2:3]
    b0 = duc[:, 0:1]
    b1 = duc[:, 1:2]
    b2 = duc[:, 2:3]
    dvi = a0 * u0 + a1 * u1 + a2 * u2
    dvj = -(b0 * u0 + b1 * u1 + b2 * u2)
    w_ij0 = a0 - dvi * u0
    w_ij1 = a1 - dvi * u1
    w_ij2 = a2 - dvi * u2
    w_ji0 = b0 + dvj * u0
    w_ji1 = b1 + dvj * u1
    w_ji2 = b2 + dvj * u2
    dih = w_ij0 * w_ji0 + w_ij1 * w_ji1 + w_ij2 * w_ji2
    dih_ref[...] = jnp.broadcast_to(dih, (dih.shape[0], H))

    wd_bf = wd_ref[...].astype(jnp.bfloat16).astype(jnp.float32)
    colsum_d = jnp.sum(wd_bf, axis=0, keepdims=True)
    dih_bf = dih.astype(jnp.bfloat16).astype(jnp.float32)
    dmod = jax.nn.sigmoid(dih_bf * colsum_d + bd_ref[...])
    f = f_ref[...]
    fout_ref[...] = f + (jnp.dot(f, we_ref[...], preferred_element_type=jnp.float32)
                         + be_ref[...]) * dmod


def _node_body(h_ref, du_ref, v_ref, vu_ref, ws_ref, bs_ref, wa_ref, ba_ref,
               hout_ref, ang_ref, vout_ref):
    du = du_ref[...]
    ang = du[:, 0:1] ** 2 + du[:, 1:2] ** 2 + du[:, 2:3] ** 2
    ang_ref[...] = jnp.broadcast_to(ang, (ang.shape[0], H))
    wa_bf = wa_ref[...].astype(jnp.bfloat16).astype(jnp.float32)
    colsum_a = jnp.sum(wa_bf, axis=0, keepdims=True)
    ang_bf = ang.astype(jnp.bfloat16).astype(jnp.float32)
    amod = jax.nn.sigmoid(ang_bf * colsum_a + ba_ref[...])
    h = h_ref[...]
    hout_ref[...] = h + (jnp.dot(h, ws_ref[...], preferred_element_type=jnp.float32)
                         + bs_ref[...]) * amod
    v = v_ref[...]
    for s in range(NSLICE):
        vout_ref[:, SW * s:SW * (s + 1)] = v[:, SW * s:SW * (s + 1)] + vu_ref[s]


def kernel(h, v, f, pos, edge_index, edge_rbf,
           lin_msg_w, lin_msg_b, lin_vec_w, lin_vec_b,
           lin_scalar_w, lin_scalar_b, lin_edge_w, lin_edge_b,
           lin_angular_w, lin_angular_b, lin_dihedral_w, lin_dihedral_b):
    n = pos.shape[0]
    e = edge_index.shape[1]
    row = edge_index[0]
    col = edge_index[1]

    pos8 = jnp.pad(pos, ((0, 0), (0, 125)))
    pr8, pc8 = _sc_pos_gather_call(pos8, row, col)
    v2 = v.reshape(n, 3 * H)
    hc, hr, vr = _sc_hv_gather_call(h, v2, row, col)

    geom, negu = pl.pallas_call(
        _geom_body,
        grid=(e // EB,),
        in_specs=[pl.BlockSpec((EB, 128), lambda i: (i, 0)),
                  pl.BlockSpec((EB, 128), lambda i: (i, 0))],
        out_specs=[pl.BlockSpec((EB, 128), lambda i: (i, 0)),
                   pl.BlockSpec((EB, 128), lambda i: (i, 0))],
        out_shape=[jax.ShapeDtypeStruct((e, 128), jnp.float32),
                   jax.ShapeDtypeStruct((e, 128), jnp.float32)],
    )(pc8, pr8)

    zeros_n8 = jnp.zeros((n, 128), jnp.float32)
    du8 = _sc_du_scatter_call(geom, negu, row, col, zeros_n8, n)
    dur8, duc8 = _sc_du_gather_call(du8, row, col)
    du = du8[:, :3]

    rbf_p = jnp.pad(edge_rbf, ((0, 0), (0, RP - R)))
    w1a = lin_msg_w[:H]
    w1b = lin_msg_w[H:2 * H]
    w1c = jnp.pad(lin_msg_w[2 * H:], ((0, RP - R), (0, 0)))
    b1 = lin_msg_b.reshape(1, H)
    bv = lin_vec_b.reshape(1, 2 * H)
    be = lin_edge_b.reshape(1, H)
    bd = lin_dihedral_b.reshape(1, H)
    bs = lin_scalar_b.reshape(1, H)
    ba = lin_angular_b.reshape(1, H)

    wspec = pl.BlockSpec(None, lambda i: (0, 0))
    espec = lambda w: pl.BlockSpec((EB, w), lambda i: (i, 0))
    msg6, f_updated, dihedral_info = pl.pallas_call(
        _edge_body,
        grid=(e // EB,),
        in_specs=[espec(H), espec(H), espec(RP), espec(3 * H), espec(H),
                  espec(128), espec(128), espec(128),
                  wspec, wspec, wspec, wspec, wspec, wspec,
                  wspec, wspec, wspec, wspec],
        out_specs=[pl.BlockSpec((NSLICE, EB, SW), lambda i: (0, i, 0)),
                   espec(H), espec(H)],
        out_shape=[jax.ShapeDtypeStruct((NSLICE, e, SW), jnp.float32),
                   jax.ShapeDtypeStruct((e, H), jnp.float32),
                   jax.ShapeDtypeStruct((e, H), jnp.float32)],
    )(hc, hr, rbf_p, vr, f, geom, dur8, duc8,
      w1a, w1b, w1c, b1, lin_vec_w, bv,
      lin_edge_w, be, lin_dihedral_w, bd)

    zeros_nw = jnp.zeros((n, SW), jnp.float32)
    vupd6 = _sc_scatter_call(msg6, col, zeros_nw, n)

    du_p = du8
    h_updated, angular_info, v_updated = pl.pallas_call(
        _node_body,
        grid=(n // NB,),
        in_specs=[pl.BlockSpec((NB, H), lambda i: (i, 0)),
                  pl.BlockSpec((NB, 128), lambda i: (i, 0)),
                  pl.BlockSpec((NB, 3 * H), lambda i: (i, 0)),
                  pl.BlockSpec((NSLICE, NB, SW), lambda i: (0, i, 0)),
                  wspec, wspec, wspec, wspec],
        out_specs=[pl.BlockSpec((NB, H), lambda i: (i, 0)),
                   pl.BlockSpec((NB, H), lambda i: (i, 0)),
                   pl.BlockSpec((NB, 3 * H), lambda i: (i, 0))],
        out_shape=[jax.ShapeDtypeStruct((n, H), jnp.float32),
                   jax.ShapeDtypeStruct((n, H), jnp.float32),
                   jax.ShapeDtypeStruct((n, 3 * H), jnp.float32)],
    )(h, du_p, v2, vupd6, lin_scalar_w, bs, lin_angular_w, ba)
    v_updated = v_updated.reshape(n, 3, H)

    return (h_updated, v_updated, f_updated, angular_info, dihedral_info, du)


# speedup vs baseline: 8.9347x; 1.0189x over previous
import functools

import jax
import jax.numpy as jnp
from jax import lax
from jax.experimental import pallas as pl
from jax.experimental.pallas import tpu as pltpu
from jax.experimental.pallas import tpu_sc as plsc

H = 256
R = 50
RP = 64
CUTOFF = 10.0

EB = 1000
NB = 1000

NSLICE = 6
SW = 128
TECS = 16
KB = 80


def _sc_pos_gather_call(pos8, rowi, coli):
    e = rowi.shape[0]
    NW = 2 * TECS
    per_w = e // NW
    K = 200
    nb = per_w // K
    mesh = plsc.VectorSubcoreMesh(core_axis_name="c", subcore_axis_name="s")

    @functools.partial(
        pl.kernel, mesh=mesh,
        out_type=[jax.ShapeDtypeStruct((e, 128), jnp.float32),
                  jax.ShapeDtypeStruct((e, 128), jnp.float32)],
        scratch_types=[
            pltpu.VMEM((K,), jnp.int32),
            pltpu.VMEM((K, 128), jnp.float32),
            pltpu.SemaphoreType.DMA,
        ])
    def body(pos_hbm, row_hbm, col_hbm, pr_hbm, pc_hbm, idx_v, st_v, sem):
        c = lax.axis_index("c")
        t = lax.axis_index("s")
        wbase = (t * 2 + c) * per_w

        def batch(b, _):
            base = wbase + b * K
            pltpu.sync_copy(row_hbm.at[pl.ds(base, K)], idx_v)
            pltpu.async_copy(pos_hbm.at[idx_v], st_v, sem).wait()
            pltpu.sync_copy(st_v, pr_hbm.at[pl.ds(base, K)])
            pltpu.sync_copy(col_hbm.at[pl.ds(base, K)], idx_v)
            pltpu.async_copy(pos_hbm.at[idx_v], st_v, sem).wait()
            pltpu.sync_copy(st_v, pc_hbm.at[pl.ds(base, K)])
            return 0

        lax.fori_loop(0, nb, batch, 0)

    return body(pos8, rowi, coli)


def _sc_hv_gather_call(h, v2, rowi, coli):
    e = rowi.shape[0]
    NW = 2 * TECS
    KH = 80
    nch = e // KH
    nrounds = (nch + NW - 1) // NW
    mesh = plsc.VectorSubcoreMesh(core_axis_name="c", subcore_axis_name="s")

    @functools.partial(
        pl.kernel, mesh=mesh,
        out_type=[jax.ShapeDtypeStruct((e, H), jnp.float32),
                  jax.ShapeDtypeStruct((e, H), jnp.float32),
                  jax.ShapeDtypeStruct((e, 3 * H), jnp.float32)],
        scratch_types=[
            pltpu.VMEM((KH,), jnp.int32),
            pltpu.VMEM((KH,), jnp.int32),
            pltpu.VMEM((KH, H), jnp.float32),
            pltpu.VMEM((KH, H), jnp.float32),
            pltpu.VMEM((KH, 3 * H), jnp.float32),
            pltpu.SemaphoreType.DMA,
            pltpu.SemaphoreType.DMA,
            pltpu.SemaphoreType.DMA,
            pltpu.SemaphoreType.DMA,
        ])
    def body(h_hbm, v_hbm, row_hbm, col_hbm, hc_hbm, hr_hbm, vr_hbm,
             idxc_v, idxr_v, shc_v, shr_v, sv_v, semg, semc, semr, semv):
        c = lax.axis_index("c")
        t = lax.axis_index("s")
        wid = t * 2 + c

        def batch(b, _):
            cid = b * NW + wid

            @pl.when(cid < nch)
            def _():
                base = cid * KH
                @pl.when(b > 0)
                def _():
                    pbase = (b - 1) * NW * KH + wid * KH
                    pltpu.make_async_copy(
                        shc_v, hc_hbm.at[pl.ds(pbase, KH)], semc).wait()
                    pltpu.make_async_copy(
                        shr_v, hr_hbm.at[pl.ds(pbase, KH)], semr).wait()
                    pltpu.make_async_copy(
                        sv_v, vr_hbm.at[pl.ds(pbase, KH)], semv).wait()

                pltpu.sync_copy(col_hbm.at[pl.ds(base, KH)], idxc_v)
                pltpu.sync_copy(row_hbm.at[pl.ds(base, KH)], idxr_v)
                pltpu.async_copy(h_hbm.at[idxc_v], shc_v, semg).wait()
                pltpu.async_copy(shc_v, hc_hbm.at[pl.ds(base, KH)], semc)
                pltpu.async_copy(h_hbm.at[idxr_v], shr_v, semg).wait()
                pltpu.async_copy(shr_v, hr_hbm.at[pl.ds(base, KH)], semr)
                pltpu.async_copy(v_hbm.at[idxr_v], sv_v, semg).wait()
                pltpu.async_copy(sv_v, vr_hbm.at[pl.ds(base, KH)], semv)
            return 0

        lax.fori_loop(0, nrounds, batch, 0)
        last_b = (nch - 1 - wid) // NW
        lbase = (last_b * NW + wid) * KH
        pltpu.make_async_copy(shc_v, hc_hbm.at[pl.ds(lbase, KH)], semc).wait()
        pltpu.make_async_copy(shr_v, hr_hbm.at[pl.ds(lbase, KH)], semr).wait()
        pltpu.make_async_copy(sv_v, vr_hbm.at[pl.ds(lbase, KH)], semv).wait()

    return body(h, v2, rowi, coli)


def _sc_du_scatter_call(pu, nu, rowi, coli, zeros_n8, n):
    e = rowi.shape[0]
    per_tec = e // TECS
    K = 200
    nb = per_tec // K
    RC = 80
    n_chunks = n // RC
    k_rounds = (n_chunks + TECS - 1) // TECS
    mesh = plsc.VectorSubcoreMesh(core_axis_name="c", subcore_axis_name="s")

    @functools.partial(
        pl.kernel, mesh=mesh,
        out_type=jax.ShapeDtypeStruct((n, 128), jnp.float32),
        scratch_types=[
            pltpu.VMEM((K,), jnp.int32),
            pltpu.VMEM((K, 128), jnp.float32),
            pltpu.VMEM_SHARED((n, 128), jnp.float32),
        ])
    def body(pu_hbm, nu_hbm, row_hbm, col_hbm, zero_hbm, out_hbm,
             idx_v, stage_v, acc_sh):
        c = lax.axis_index("c")
        t = lax.axis_index("s")

        @pl.when(c == 0)
        def _():
            for k in range(k_rounds):
                cid = k * TECS + t

                @pl.when(cid < n_chunks)
                def _():
                    pltpu.sync_copy(zero_hbm.at[pl.ds(cid * RC, RC)],
                                    acc_sh.at[pl.ds(cid * RC, RC)])
            plsc.subcore_barrier()
            ebase = t * per_tec

            def batch(b, _):
                base = ebase + b * K
                pltpu.sync_copy(row_hbm.at[pl.ds(base, K)], idx_v)
                pltpu.sync_copy(pu_hbm.at[pl.ds(base, K)], stage_v)
                pltpu.sync_copy(stage_v, acc_sh.at[idx_v], add=True)
                pltpu.sync_copy(col_hbm.at[pl.ds(base, K)], idx_v)
                pltpu.sync_copy(nu_hbm.at[pl.ds(base, K)], stage_v)
                pltpu.sync_copy(stage_v, acc_sh.at[idx_v], add=True)
                return 0

            lax.fori_loop(0, nb, batch, 0)
            plsc.subcore_barrier()
            for k in range(k_rounds):
                cid = k * TECS + t

                @pl.when(cid < n_chunks)
                def _():
                    pltpu.sync_copy(acc_sh.at[pl.ds(cid * RC, RC)],
                                    out_hbm.at[pl.ds(cid * RC, RC)])

    return body(pu, nu, rowi, coli, zeros_n8)


def _sc_du_gather_call(du8, rowi, coli):
    e = rowi.shape[0]
    NW = 2 * TECS
    per_w = e // NW
    K = 200
    nb = per_w // K
    mesh = plsc.VectorSubcoreMesh(core_axis_name="c", subcore_axis_name="s")

    @functools.partial(
        pl.kernel, mesh=mesh,
        out_type=[jax.ShapeDtypeStruct((e, 128), jnp.float32),
                  jax.ShapeDtypeStruct((e, 128), jnp.float32)],
        scratch_types=[
            pltpu.VMEM((K,), jnp.int32),
            pltpu.VMEM((K, 128), jnp.float32),
            pltpu.SemaphoreType.DMA,
        ])
    def body(du_hbm, row_hbm, col_hbm, dur_hbm, duc_hbm, idx_v, st_v, sem):
        c = lax.axis_index("c")
        t = lax.axis_index("s")
        wbase = (t * 2 + c) * per_w

        def batch(b, _):
            base = wbase + b * K
            pltpu.sync_copy(row_hbm.at[pl.ds(base, K)], idx_v)
            pltpu.async_copy(du_hbm.at[idx_v], st_v, sem).wait()
            pltpu.sync_copy(st_v, dur_hbm.at[pl.ds(base, K)])
            pltpu.sync_copy(col_hbm.at[pl.ds(base, K)], idx_v)
            pltpu.async_copy(du_hbm.at[idx_v], st_v, sem).wait()
            pltpu.sync_copy(st_v, duc_hbm.at[pl.ds(base, K)])
            return 0

        lax.fori_loop(0, nb, batch, 0)

    return body(du8, rowi, coli)


def _sc_scatter_call(msg6, col, zeros_nw, n):
    e = col.shape[0]
    per_tec = e // TECS
    nb = per_tec // KB
    RC = 80
    n_chunks = n // RC
    k_rounds = (n_chunks + TECS - 1) // TECS
    mesh = plsc.VectorSubcoreMesh(core_axis_name="c", subcore_axis_name="s")

    @functools.partial(
        pl.kernel, mesh=mesh,
        out_type=jax.ShapeDtypeStruct((NSLICE, n, SW), jnp.float32),
        scratch_types=[
            pltpu.VMEM((KB,), jnp.int32),
            pltpu.VMEM((KB,), jnp.int32),
            pltpu.VMEM((KB, SW), jnp.float32),
            pltpu.VMEM((KB, SW), jnp.float32),
            pltpu.VMEM_SHARED((n, SW), jnp.float32),
            pltpu.SemaphoreType.DMA,
            pltpu.SemaphoreType.DMA,
            pltpu.SemaphoreType.DMA,
            pltpu.SemaphoreType.DMA,
        ])
    def body(msg_hbm, col_hbm, zero_hbm, out_hbm,
             idx0_v, idx1_v, st0_v, st1_v, acc_sh,
             semi0, semi1, semm0, semm1):
        c = lax.axis_index("c")
        t = lax.axis_index("s")
        ebase = t * per_tec
        idx_bufs = (idx0_v, idx1_v)
        st_bufs = (st0_v, st1_v)
        sems_i = (semi0, semi1)
        sems_m = (semm0, semm1)

        def load(b, j):
            base = ebase + b * KB
            pltpu.async_copy(col_hbm.at[pl.ds(base, KB)], idx_bufs[j],
                             sems_i[j])
            pltpu.async_copy(msg_hbm.at[s, pl.ds(base, KB)], st_bufs[j],
                             sems_m[j])

        def drain_scatter(b, j):
            base = ebase + b * KB
            pltpu.make_async_copy(col_hbm.at[pl.ds(base, KB)],
                                  idx_bufs[j], sems_i[j]).wait()
            pltpu.make_async_copy(msg_hbm.at[s, pl.ds(base, KB)],
                                  st_bufs[j], sems_m[j]).wait()
            pltpu.sync_copy(st_bufs[j], acc_sh.at[idx_bufs[j]], add=True)

        for s_local in range(NSLICE // 2):
            s = c * (NSLICE // 2) + s_local
            for k in range(k_rounds):
                cid = k * TECS + t

                @pl.when(cid < n_chunks)
                def _():
                    pltpu.sync_copy(zero_hbm.at[pl.ds(cid * RC, RC)],
                                    acc_sh.at[pl.ds(cid * RC, RC)])
            plsc.subcore_barrier()

            load(0, 0)

            def pair(k2, _):
                b0 = 2 * k2
                load(b0 + 1, 1)
                drain_scatter(b0, 0)

                @pl.when(b0 + 2 < nb)
                def _():
                    load(b0 + 2, 0)
                drain_scatter(b0 + 1, 1)
                return 0

            lax.fori_loop(0, nb // 2, pair, 0)
            if nb % 2 == 1:
                drain_scatter(nb - 1, 0)
            plsc.subcore_barrier()
            for k in range(k_rounds):
                cid = k * TECS + t

                @pl.when(cid < n_chunks)
                def _():
                    pltpu.sync_copy(acc_sh.at[pl.ds(cid * RC, RC)],
                                    out_hbm.at[s, pl.ds(cid * RC, RC)])
            plsc.subcore_barrier()

    return body(msg6, col, zeros_nw)


def _geom_body(pc_ref, pr_ref, geom_ref, negu_ref):
    pc = pc_ref[...]
    pr = pr_ref[...]
    ev = pc - pr
    e0 = ev[:, 0:1]
    e1 = ev[:, 1:2]
    e2 = ev[:, 2:3]
    dist = jnp.sqrt(e0 * e0 + e1 * e1 + e2 * e2) + 1e-8
    inv = 1.0 / dist
    u0 = e0 * inv
    u1 = e1 * inv
    u2 = e2 * inv
    cw = 0.5 * (jnp.cos(jnp.pi * dist / CUTOFF) + 1.0)
    cw = jnp.where(dist < CUTOFF, cw, 0.0)
    zero = jnp.zeros((u0.shape[0], 124), jnp.float32)
    geom_ref[...] = jnp.concatenate([u0, u1, u2, cw, zero], axis=1)
    negu_ref[...] = jnp.concatenate(
        [-u0, -u1, -u2, jnp.zeros_like(u0), zero], axis=1)


def _edge_body(hc_ref, hr_ref, rbf_ref, vr_ref, f_ref, geom_ref,
               dur_ref, duc_ref,
               w1a_ref, w1b_ref, w1c_ref, b1_ref, wv_ref, bv_ref,
               we_ref, be_ref, wd_ref, bd_ref,
               msg_ref, fout_ref, dih_ref):
    hc = hc_ref[...]
    hr = hr_ref[...]
    rbf = rbf_ref[...]
    geom = geom_ref[...]
    u0 = geom[:, 0:1]
    u1 = geom[:, 1:2]
    u2 = geom[:, 2:3]
    cw = geom[:, 3:4]

    sm = (jnp.dot(hc, w1a_ref[...], preferred_element_type=jnp.float32)
          + jnp.dot(hr, w1b_ref[...], preferred_element_type=jnp.float32)
          + jnp.dot(rbf, w1c_ref[...], preferred_element_type=jnp.float32)
          + b1_ref[...])
    vw = jnp.dot(sm, wv_ref[...], preferred_element_type=jnp.float32) + bv_ref[...]
    w1 = vw[:, :H] * cw
    w2 = vw[:, H:] * cw
    vr = vr_ref[...]
    us = (u0, u1, u2)
    for s in range(NSLICE):
        d, half = s // 2, s % 2
        msg_ref[s] = (w1[:, SW * half:SW * (half + 1)] * us[d]
                      + w2[:, SW * half:SW * (half + 1)]
                      * vr[:, H * d + SW * half:H * d + SW * (half + 1)])

    dur = dur_ref[...]
    duc = duc_ref[...]
    a0 = dur[:, 0:1]
    a1 = dur[:, 1:2]
    a2 = dur[:, 2:3]
    b0 = duc[:, 0:1]
    b1 = duc[:, 1:2]
    b2 = duc[:, 2:3]
    dvi = a0 * u0 + a1 * u1 + a2 * u2
    dvj = -(b0 * u0 + b1 * u1 + b2 * u2)
    w_ij0 = a0 - dvi * u0
    w_ij1 = a1 - dvi * u1
    w_ij2 = a2 - dvi * u2
    w_ji0 = b0 + dvj * u0
    w_ji1 = b1 + dvj * u1
    w_ji2 = b2 + dvj * u2
    dih = w_ij0 * w_ji0 + w_ij1 * w_ji1 + w_ij2 * w_ji2
    dih_ref[...] = jnp.broadcast_to(dih, (dih.shape[0], H))

    wd_bf = wd_ref[...].astype(jnp.bfloat16).astype(jnp.float32)
    colsum_d = jnp.sum(wd_bf, axis=0, keepdims=True)
    dih_bf = dih.astype(jnp.bfloat16).astype(jnp.float32)
    dmod = jax.nn.sigmoid(dih_bf * colsum_d + bd_ref[...])
    f = f_ref[...]
    fout_ref[...] = f + (jnp.dot(f, we_ref[...], preferred_element_type=jnp.float32)
                         + be_ref[...]) * dmod


def _node_body(h_ref, du_ref, v_ref, vu_ref, ws_ref, bs_ref, wa_ref, ba_ref,
               hout_ref, ang_ref, vout_ref):
    du = du_ref[...]
    ang = du[:, 0:1] ** 2 + du[:, 1:2] ** 2 + du[:, 2:3] ** 2
    ang_ref[...] = jnp.broadcast_to(ang, (ang.shape[0], H))
    wa_bf = wa_ref[...].astype(jnp.bfloat16).astype(jnp.float32)
    colsum_a = jnp.sum(wa_bf, axis=0, keepdims=True)
    ang_bf = ang.astype(jnp.bfloat16).astype(jnp.float32)
    amod = jax.nn.sigmoid(ang_bf * colsum_a + ba_ref[...])
    h = h_ref[...]
    hout_ref[...] = h + (jnp.dot(h, ws_ref[...], preferred_element_type=jnp.float32)
                         + bs_ref[...]) * amod
    v = v_ref[...]
    for s in range(NSLICE):
        vout_ref[:, SW * s:SW * (s + 1)] = v[:, SW * s:SW * (s + 1)] + vu_ref[s]


def kernel(h, v, f, pos, edge_index, edge_rbf,
           lin_msg_w, lin_msg_b, lin_vec_w, lin_vec_b,
           lin_scalar_w, lin_scalar_b, lin_edge_w, lin_edge_b,
           lin_angular_w, lin_angular_b, lin_dihedral_w, lin_dihedral_b):
    n = pos.shape[0]
    e = edge_index.shape[1]
    row = edge_index[0]
    col = edge_index[1]

    pos8 = jnp.pad(pos, ((0, 0), (0, 125)))
    pr8, pc8 = _sc_pos_gather_call(pos8, row, col)
    v2 = v.reshape(n, 3 * H)
    hc, hr, vr = _sc_hv_gather_call(h, v2, row, col)

    geom, negu = pl.pallas_call(
        _geom_body,
        grid=(e // EB,),
        in_specs=[pl.BlockSpec((EB, 128), lambda i: (i, 0)),
                  pl.BlockSpec((EB, 128), lambda i: (i, 0))],
        out_specs=[pl.BlockSpec((EB, 128), lambda i: (i, 0)),
                   pl.BlockSpec((EB, 128), lambda i: (i, 0))],
        out_shape=[jax.ShapeDtypeStruct((e, 128), jnp.float32),
                   jax.ShapeDtypeStruct((e, 128), jnp.float32)],
    )(pc8, pr8)

    zeros_n8 = jnp.zeros((n, 128), jnp.float32)
    du8 = _sc_du_scatter_call(geom, negu, row, col, zeros_n8, n)
    dur8, duc8 = _sc_du_gather_call(du8, row, col)
    du = du8[:, :3]

    rbf_p = jnp.pad(edge_rbf, ((0, 0), (0, RP - R)))
    w1a = lin_msg_w[:H]
    w1b = lin_msg_w[H:2 * H]
    w1c = jnp.pad(lin_msg_w[2 * H:], ((0, RP - R), (0, 0)))
    b1 = lin_msg_b.reshape(1, H)
    bv = lin_vec_b.reshape(1, 2 * H)
    be = lin_edge_b.reshape(1, H)
    bd = lin_dihedral_b.reshape(1, H)
    bs = lin_scalar_b.reshape(1, H)
    ba = lin_angular_b.reshape(1, H)

    wspec = pl.BlockSpec(None, lambda i: (0, 0))
    espec = lambda w: pl.BlockSpec((EB, w), lambda i: (i, 0))
    msg6, f_updated, dihedral_info = pl.pallas_call(
        _edge_body,
        grid=(e // EB,),
        in_specs=[espec(H), espec(H), espec(RP), espec(3 * H), espec(H),
                  espec(128), espec(128), espec(128),
                  wspec, wspec, wspec, wspec, wspec, wspec,
                  wspec, wspec, wspec, wspec],
        out_specs=[pl.BlockSpec((NSLICE, EB, SW), lambda i: (0, i, 0)),
                   espec(H), espec(H)],
        out_shape=[jax.ShapeDtypeStruct((NSLICE, e, SW), jnp.float32),
                   jax.ShapeDtypeStruct((e, H), jnp.float32),
                   jax.ShapeDtypeStruct((e, H), jnp.float32)],
    )(hc, hr, rbf_p, vr, f, geom, dur8, duc8,
      w1a, w1b, w1c, b1, lin_vec_w, bv,
      lin_edge_w, be, lin_dihedral_w, bd)

    zeros_nw = jnp.zeros((n, SW), jnp.float32)
    vupd6 = _sc_scatter_call(msg6, col, zeros_nw, n)

    du_p = du8
    h_updated, angular_info, v_updated = pl.pallas_call(
        _node_body,
        grid=(n // NB,),
        in_specs=[pl.BlockSpec((NB, H), lambda i: (i, 0)),
                  pl.BlockSpec((NB, 128), lambda i: (i, 0)),
                  pl.BlockSpec((NB, 3 * H), lambda i: (i, 0)),
                  pl.BlockSpec((NSLICE, NB, SW), lambda i: (0, i, 0)),
                  wspec, wspec, wspec, wspec],
        out_specs=[pl.BlockSpec((NB, H), lambda i: (i, 0)),
                   pl.BlockSpec((NB, H), lambda i: (i, 0)),
                   pl.BlockSpec((NB, 3 * H), lambda i: (i, 0))],
        out_shape=[jax.ShapeDtypeStruct((n, H), jnp.float32),
                   jax.ShapeDtypeStruct((n, H), jnp.float32),
                   jax.ShapeDtypeStruct((n, 3 * H), jnp.float32)],
    )(h, du_p, v2, vupd6, lin_scalar_w, bs, lin_angular_w, ba)
    v_updated = v_updated.reshape(n, 3, H)

    return (h_updated, v_updated, f_updated, angular_info, dihedral_info, du)


# speedup vs baseline: 10.5719x; 1.1832x over previous
import functools

import jax
import jax.numpy as jnp
from jax import lax
from jax.experimental import pallas as pl
from jax.experimental.pallas import tpu as pltpu
from jax.experimental.pallas import tpu_sc as plsc

H = 256
R = 50
RP = 64
CUTOFF = 10.0

EB = 1000
NB = 1000

NSLICE = 6
SW = 128
TECS = 16
KB = 80


def _sc_pos_gather_call(pos8, rowi, coli):
    e = rowi.shape[0]
    NW = 2 * TECS
    per_w = e // NW
    K = 200
    nb = per_w // K
    mesh = plsc.VectorSubcoreMesh(core_axis_name="c", subcore_axis_name="s")

    @functools.partial(
        pl.kernel, mesh=mesh,
        out_type=[jax.ShapeDtypeStruct((e, 128), jnp.float32),
                  jax.ShapeDtypeStruct((e, 128), jnp.float32)],
        scratch_types=[
            pltpu.VMEM((K,), jnp.int32),
            pltpu.VMEM((K, 128), jnp.float32),
            pltpu.SemaphoreType.DMA,
        ])
    def body(pos_hbm, row_hbm, col_hbm, pr_hbm, pc_hbm, idx_v, st_v, sem):
        c = lax.axis_index("c")
        t = lax.axis_index("s")
        wbase = (t * 2 + c) * per_w

        def batch(b, _):
            base = wbase + b * K
            pltpu.sync_copy(row_hbm.at[pl.ds(base, K)], idx_v)
            pltpu.async_copy(pos_hbm.at[idx_v], st_v, sem).wait()
            pltpu.sync_copy(st_v, pr_hbm.at[pl.ds(base, K)])
            pltpu.sync_copy(col_hbm.at[pl.ds(base, K)], idx_v)
            pltpu.async_copy(pos_hbm.at[idx_v], st_v, sem).wait()
            pltpu.sync_copy(st_v, pc_hbm.at[pl.ds(base, K)])
            return 0

        lax.fori_loop(0, nb, batch, 0)

    return body(pos8, rowi, coli)


def _sc_hv_gather_call(h, v2, rowi, coli):
    e = rowi.shape[0]
    NW = 2 * TECS
    KH = 80
    nch = e // KH
    nrounds = (nch + NW - 1) // NW
    mesh = plsc.VectorSubcoreMesh(core_axis_name="c", subcore_axis_name="s")

    @functools.partial(
        pl.kernel, mesh=mesh,
        out_type=[jax.ShapeDtypeStruct((e, H), jnp.float32),
                  jax.ShapeDtypeStruct((e, H), jnp.float32),
                  jax.ShapeDtypeStruct((e, 3 * H), jnp.float32)],
        scratch_types=[
            pltpu.VMEM((KH,), jnp.int32),
            pltpu.VMEM((KH,), jnp.int32),
            pltpu.VMEM((KH, H), jnp.float32),
            pltpu.VMEM((KH, H), jnp.float32),
            pltpu.VMEM((KH, 3 * H), jnp.float32),
            pltpu.SemaphoreType.DMA,
            pltpu.SemaphoreType.DMA,
            pltpu.SemaphoreType.DMA,
            pltpu.SemaphoreType.DMA,
        ])
    def body(h_hbm, v_hbm, row_hbm, col_hbm, hc_hbm, hr_hbm, vr_hbm,
             idxc_v, idxr_v, shc_v, shr_v, sv_v, semg, semc, semr, semv):
        c = lax.axis_index("c")
        t = lax.axis_index("s")
        wid = t * 2 + c

        def batch(b, _):
            cid = b * NW + wid

            @pl.when(cid < nch)
            def _():
                base = cid * KH
                @pl.when(b > 0)
                def _():
                    pbase = (b - 1) * NW * KH + wid * KH
                    pltpu.make_async_copy(
                        shc_v, hc_hbm.at[pl.ds(pbase, KH)], semc).wait()
                    pltpu.make_async_copy(
                        shr_v, hr_hbm.at[pl.ds(pbase, KH)], semr).wait()
                    pltpu.make_async_copy(
                        sv_v, vr_hbm.at[pl.ds(pbase, KH)], semv).wait()

                pltpu.sync_copy(col_hbm.at[pl.ds(base, KH)], idxc_v)
                pltpu.sync_copy(row_hbm.at[pl.ds(base, KH)], idxr_v)
                pltpu.async_copy(h_hbm.at[idxc_v], shc_v, semg).wait()
                pltpu.async_copy(shc_v, hc_hbm.at[pl.ds(base, KH)], semc)
                pltpu.async_copy(h_hbm.at[idxr_v], shr_v, semg).wait()
                pltpu.async_copy(shr_v, hr_hbm.at[pl.ds(base, KH)], semr)
                pltpu.async_copy(v_hbm.at[idxr_v], sv_v, semg).wait()
                pltpu.async_copy(sv_v, vr_hbm.at[pl.ds(base, KH)], semv)
            return 0

        lax.fori_loop(0, nrounds, batch, 0)
        last_b = (nch - 1 - wid) // NW
        lbase = (last_b * NW + wid) * KH
        pltpu.make_async_copy(shc_v, hc_hbm.at[pl.ds(lbase, KH)], semc).wait()
        pltpu.make_async_copy(shr_v, hr_hbm.at[pl.ds(lbase, KH)], semr).wait()
        pltpu.make_async_copy(sv_v, vr_hbm.at[pl.ds(lbase, KH)], semv).wait()

    return body(h, v2, rowi, coli)


def _sc_du_scatter_call(pu, nu, rowi, coli, zeros_n8, n):
    e = rowi.shape[0]
    per_tec = e // TECS
    K = 200
    nb = per_tec // K
    RC = 80
    n_chunks = n // RC
    k_rounds = (n_chunks + TECS - 1) // TECS
    mesh = plsc.VectorSubcoreMesh(core_axis_name="c", subcore_axis_name="s")

    @functools.partial(
        pl.kernel, mesh=mesh,
        out_type=jax.ShapeDtypeStruct((n, 128), jnp.float32),
        scratch_types=[
            pltpu.VMEM((K,), jnp.int32),
            pltpu.VMEM((K, 128), jnp.float32),
            pltpu.VMEM_SHARED((n, 128), jnp.float32),
        ])
    def body(pu_hbm, nu_hbm, row_hbm, col_hbm, zero_hbm, out_hbm,
             idx_v, stage_v, acc_sh):
        c = lax.axis_index("c")
        t = lax.axis_index("s")

        @pl.when(c == 0)
        def _():
            for k in range(k_rounds):
                cid = k * TECS + t

                @pl.when(cid < n_chunks)
                def _():
                    pltpu.sync_copy(zero_hbm.at[pl.ds(cid * RC, RC)],
                                    acc_sh.at[pl.ds(cid * RC, RC)])
            plsc.subcore_barrier()
            ebase = t * per_tec

            def batch(b, _):
                base = ebase + b * K
                pltpu.sync_copy(row_hbm.at[pl.ds(base, K)], idx_v)
                pltpu.sync_copy(pu_hbm.at[pl.ds(base, K)], stage_v)
                pltpu.sync_copy(stage_v, acc_sh.at[idx_v], add=True)
                pltpu.sync_copy(col_hbm.at[pl.ds(base, K)], idx_v)
                pltpu.sync_copy(nu_hbm.at[pl.ds(base, K)], stage_v)
                pltpu.sync_copy(stage_v, acc_sh.at[idx_v], add=True)
                return 0

            lax.fori_loop(0, nb, batch, 0)
            plsc.subcore_barrier()
            for k in range(k_rounds):
                cid = k * TECS + t

                @pl.when(cid < n_chunks)
                def _():
                    pltpu.sync_copy(acc_sh.at[pl.ds(cid * RC, RC)],
                                    out_hbm.at[pl.ds(cid * RC, RC)])

    return body(pu, nu, rowi, coli, zeros_n8)


def _sc_du_gather_call(du8, rowi, coli):
    e = rowi.shape[0]
    NW = 2 * TECS
    per_w = e // NW
    K = 200
    nb = per_w // K
    mesh = plsc.VectorSubcoreMesh(core_axis_name="c", subcore_axis_name="s")

    @functools.partial(
        pl.kernel, mesh=mesh,
        out_type=[jax.ShapeDtypeStruct((e, 128), jnp.float32),
                  jax.ShapeDtypeStruct((e, 128), jnp.float32)],
        scratch_types=[
            pltpu.VMEM((K,), jnp.int32),
            pltpu.VMEM((K, 128), jnp.float32),
            pltpu.SemaphoreType.DMA,
        ])
    def body(du_hbm, row_hbm, col_hbm, dur_hbm, duc_hbm, idx_v, st_v, sem):
        c = lax.axis_index("c")
        t = lax.axis_index("s")
        wbase = (t * 2 + c) * per_w

        def batch(b, _):
            base = wbase + b * K
            pltpu.sync_copy(row_hbm.at[pl.ds(base, K)], idx_v)
            pltpu.async_copy(du_hbm.at[idx_v], st_v, sem).wait()
            pltpu.sync_copy(st_v, dur_hbm.at[pl.ds(base, K)])
            pltpu.sync_copy(col_hbm.at[pl.ds(base, K)], idx_v)
            pltpu.async_copy(du_hbm.at[idx_v], st_v, sem).wait()
            pltpu.sync_copy(st_v, duc_hbm.at[pl.ds(base, K)])
            return 0

        lax.fori_loop(0, nb, batch, 0)

    return body(du8, rowi, coli)


def _sc_scatter_call(msg6, col, zeros_nw, n):
    e = col.shape[0]
    per_tec = e // TECS
    nb = per_tec // KB
    RC = 80
    n_chunks = n // RC
    k_rounds = (n_chunks + TECS - 1) // TECS
    mesh = plsc.VectorSubcoreMesh(core_axis_name="c", subcore_axis_name="s")

    @functools.partial(
        pl.kernel, mesh=mesh,
        out_type=jax.ShapeDtypeStruct((NSLICE, n, SW), jnp.float32),
        scratch_types=[
            pltpu.VMEM((KB,), jnp.int32),
            pltpu.VMEM((KB,), jnp.int32),
            pltpu.VMEM((KB, SW), jnp.float32),
            pltpu.VMEM((KB, SW), jnp.float32),
            pltpu.VMEM_SHARED((n, SW), jnp.float32),
            pltpu.SemaphoreType.DMA,
            pltpu.SemaphoreType.DMA,
            pltpu.SemaphoreType.DMA,
            pltpu.SemaphoreType.DMA,
        ])
    def body(msg_hbm, col_hbm, zero_hbm, out_hbm,
             idx0_v, idx1_v, st0_v, st1_v, acc_sh,
             semi0, semi1, semm0, semm1):
        c = lax.axis_index("c")
        t = lax.axis_index("s")
        ebase = t * per_tec
        idx_bufs = (idx0_v, idx1_v)
        st_bufs = (st0_v, st1_v)
        sems_i = (semi0, semi1)
        sems_m = (semm0, semm1)

        def load(b, j):
            base = ebase + b * KB
            pltpu.async_copy(col_hbm.at[pl.ds(base, KB)], idx_bufs[j],
                             sems_i[j])
            pltpu.async_copy(msg_hbm.at[s, pl.ds(base, KB)], st_bufs[j],
                             sems_m[j])

        def drain_scatter(b, j):
            base = ebase + b * KB
            pltpu.make_async_copy(col_hbm.at[pl.ds(base, KB)],
                                  idx_bufs[j], sems_i[j]).wait()
            pltpu.make_async_copy(msg_hbm.at[s, pl.ds(base, KB)],
                                  st_bufs[j], sems_m[j]).wait()
            pltpu.sync_copy(st_bufs[j], acc_sh.at[idx_bufs[j]], add=True)

        for s_local in range(NSLICE // 2):
            s = c * (NSLICE // 2) + s_local
            for k in range(k_rounds):
                cid = k * TECS + t

                @pl.when(cid < n_chunks)
                def _():
                    pltpu.sync_copy(zero_hbm.at[pl.ds(cid * RC, RC)],
                                    acc_sh.at[pl.ds(cid * RC, RC)])
            plsc.subcore_barrier()

            load(0, 0)

            def pair(k2, _):
                b0 = 2 * k2
                load(b0 + 1, 1)
                drain_scatter(b0, 0)

                @pl.when(b0 + 2 < nb)
                def _():
                    load(b0 + 2, 0)
                drain_scatter(b0 + 1, 1)
                return 0

            lax.fori_loop(0, nb // 2, pair, 0)
            if nb % 2 == 1:
                drain_scatter(nb - 1, 0)
            plsc.subcore_barrier()
            for k in range(k_rounds):
                cid = k * TECS + t

                @pl.when(cid < n_chunks)
                def _():
                    pltpu.sync_copy(acc_sh.at[pl.ds(cid * RC, RC)],
                                    out_hbm.at[s, pl.ds(cid * RC, RC)])
            plsc.subcore_barrier()

    return body(msg6, col, zeros_nw)


def _geom_body(pc_ref, pr_ref, geom_ref, negu_ref):
    pc = pc_ref[...]
    pr = pr_ref[...]
    ev = pc - pr
    e0 = ev[:, 0:1]
    e1 = ev[:, 1:2]
    e2 = ev[:, 2:3]
    dist = jnp.sqrt(e0 * e0 + e1 * e1 + e2 * e2) + 1e-8
    inv = 1.0 / dist
    u0 = e0 * inv
    u1 = e1 * inv
    u2 = e2 * inv
    cw = 0.5 * (jnp.cos(jnp.pi * dist / CUTOFF) + 1.0)
    cw = jnp.where(dist < CUTOFF, cw, 0.0)
    zero = jnp.zeros((u0.shape[0], 124), jnp.float32)
    geom_ref[...] = jnp.concatenate([u0, u1, u2, cw, zero], axis=1)
    negu_ref[...] = jnp.concatenate(
        [-u0, -u1, -u2, jnp.zeros_like(u0), zero], axis=1)


def _msg_body(hc_ref, hr_ref, rbf_ref, vr_ref, geom_ref,
              w1a_ref, w1b_ref, w1c_ref, b1_ref, wv_ref, bv_ref,
              msg_ref):
    hc = hc_ref[...]
    hr = hr_ref[...]
    rbf = rbf_ref[...]
    geom = geom_ref[...]
    u0 = geom[:, 0:1]
    u1 = geom[:, 1:2]
    u2 = geom[:, 2:3]
    cw = geom[:, 3:4]

    sm = (jnp.dot(hc, w1a_ref[...], preferred_element_type=jnp.float32)
          + jnp.dot(hr, w1b_ref[...], preferred_element_type=jnp.float32)
          + jnp.dot(rbf, w1c_ref[...], preferred_element_type=jnp.float32)
          + b1_ref[...])
    vw = jnp.dot(sm, wv_ref[...], preferred_element_type=jnp.float32) + bv_ref[...]
    w1 = vw[:, :H] * cw
    w2 = vw[:, H:] * cw
    vr = vr_ref[...]
    us = (u0, u1, u2)
    for s in range(NSLICE):
        d, half = s // 2, s % 2
        msg_ref[s] = (w1[:, SW * half:SW * (half + 1)] * us[d]
                      + w2[:, SW * half:SW * (half + 1)]
                      * vr[:, H * d + SW * half:H * d + SW * (half + 1)])


def _fup_body(f_ref, geom_ref, dur_ref, duc_ref,
              we_ref, be_ref, wd_ref, bd_ref,
              fout_ref, dih_ref):
    geom = geom_ref[...]
    u0 = geom[:, 0:1]
    u1 = geom[:, 1:2]
    u2 = geom[:, 2:3]
    dur = dur_ref[...]
    duc = duc_ref[...]
    a0 = dur[:, 0:1]
    a1 = dur[:, 1:2]
    a2 = dur[:, 2:3]
    b0 = duc[:, 0:1]
    b1 = duc[:, 1:2]
    b2 = duc[:, 2:3]
    dvi = a0 * u0 + a1 * u1 + a2 * u2
    dvj = -(b0 * u0 + b1 * u1 + b2 * u2)
    w_ij0 = a0 - dvi * u0
    w_ij1 = a1 - dvi * u1
    w_ij2 = a2 - dvi * u2
    w_ji0 = b0 + dvj * u0
    w_ji1 = b1 + dvj * u1
    w_ji2 = b2 + dvj * u2
    dih = w_ij0 * w_ji0 + w_ij1 * w_ji1 + w_ij2 * w_ji2
    dih_ref[...] = jnp.broadcast_to(dih, (dih.shape[0], H))

    wd_bf = wd_ref[...].astype(jnp.bfloat16).astype(jnp.float32)
    colsum_d = jnp.sum(wd_bf, axis=0, keepdims=True)
    dih_bf = dih.astype(jnp.bfloat16).astype(jnp.float32)
    dmod = jax.nn.sigmoid(dih_bf * colsum_d + bd_ref[...])
    f = f_ref[...]
    fout_ref[...] = f + (jnp.dot(f, we_ref[...], preferred_element_type=jnp.float32)
                         + be_ref[...]) * dmod


def _node_body(h_ref, du_ref, v_ref, vu_ref, ws_ref, bs_ref, wa_ref, ba_ref,
               hout_ref, ang_ref, vout_ref):
    du = du_ref[...]
    ang = du[:, 0:1] ** 2 + du[:, 1:2] ** 2 + du[:, 2:3] ** 2
    ang_ref[...] = jnp.broadcast_to(ang, (ang.shape[0], H))
    wa_bf = wa_ref[...].astype(jnp.bfloat16).astype(jnp.float32)
    colsum_a = jnp.sum(wa_bf, axis=0, keepdims=True)
    ang_bf = ang.astype(jnp.bfloat16).astype(jnp.float32)
    amod = jax.nn.sigmoid(ang_bf * colsum_a + ba_ref[...])
    h = h_ref[...]
    hout_ref[...] = h + (jnp.dot(h, ws_ref[...], preferred_element_type=jnp.float32)
                         + bs_ref[...]) * amod
    v = v_ref[...]
    for s in range(NSLICE):
        vout_ref[:, SW * s:SW * (s + 1)] = v[:, SW * s:SW * (s + 1)] + vu_ref[s]


def kernel(h, v, f, pos, edge_index, edge_rbf,
           lin_msg_w, lin_msg_b, lin_vec_w, lin_vec_b,
           lin_scalar_w, lin_scalar_b, lin_edge_w, lin_edge_b,
           lin_angular_w, lin_angular_b, lin_dihedral_w, lin_dihedral_b):
    n = pos.shape[0]
    e = edge_index.shape[1]
    row = edge_index[0]
    col = edge_index[1]

    pos8 = jnp.pad(pos, ((0, 0), (0, 125)))
    pr8, pc8 = _sc_pos_gather_call(pos8, row, col)
    v2 = v.reshape(n, 3 * H)
    hc, hr, vr = _sc_hv_gather_call(h, v2, row, col)

    geom, negu = pl.pallas_call(
        _geom_body,
        grid=(e // EB,),
        in_specs=[pl.BlockSpec((EB, 128), lambda i: (i, 0)),
                  pl.BlockSpec((EB, 128), lambda i: (i, 0))],
        out_specs=[pl.BlockSpec((EB, 128), lambda i: (i, 0)),
                   pl.BlockSpec((EB, 128), lambda i: (i, 0))],
        out_shape=[jax.ShapeDtypeStruct((e, 128), jnp.float32),
                   jax.ShapeDtypeStruct((e, 128), jnp.float32)],
    )(pc8, pr8)

    zeros_n8 = jnp.zeros((n, 128), jnp.float32)
    du8 = _sc_du_scatter_call(geom, negu, row, col, zeros_n8, n)
    dur8, duc8 = _sc_du_gather_call(du8, row, col)
    du = du8[:, :3]

    rbf_p = jnp.pad(edge_rbf, ((0, 0), (0, RP - R)))
    w1a = lin_msg_w[:H]
    w1b = lin_msg_w[H:2 * H]
    w1c = jnp.pad(lin_msg_w[2 * H:], ((0, RP - R), (0, 0)))
    b1 = lin_msg_b.reshape(1, H)
    bv = lin_vec_b.reshape(1, 2 * H)
    be = lin_edge_b.reshape(1, H)
    bd = lin_dihedral_b.reshape(1, H)
    bs = lin_scalar_b.reshape(1, H)
    ba = lin_angular_b.reshape(1, H)

    wspec = pl.BlockSpec(None, lambda i: (0, 0))
    espec = lambda w: pl.BlockSpec((EB, w), lambda i: (i, 0))
    msg6 = pl.pallas_call(
        _msg_body,
        grid=(e // EB,),
        in_specs=[espec(H), espec(H), espec(RP), espec(3 * H),
                  espec(128),
                  wspec, wspec, wspec, wspec, wspec, wspec],
        out_specs=pl.BlockSpec((NSLICE, EB, SW), lambda i: (0, i, 0)),
        out_shape=jax.ShapeDtypeStruct((NSLICE, e, SW), jnp.float32),
    )(hc, hr, rbf_p, vr, geom,
      w1a, w1b, w1c, b1, lin_vec_w, bv)

    f_updated, dihedral_info = pl.pallas_call(
        _fup_body,
        grid=(e // EB,),
        in_specs=[espec(H), espec(128), espec(128), espec(128),
                  wspec, wspec, wspec, wspec],
        out_specs=[espec(H), espec(H)],
        out_shape=[jax.ShapeDtypeStruct((e, H), jnp.float32),
                   jax.ShapeDtypeStruct((e, H), jnp.float32)],
    )(f, geom, dur8, duc8, lin_edge_w, be, lin_dihedral_w, bd)

    zeros_nw = jnp.zeros((n, SW), jnp.float32)
    vupd6 = _sc_scatter_call(msg6, col, zeros_nw, n)

    du_p = du8
    h_updated, angular_info, v_updated = pl.pallas_call(
        _node_body,
        grid=(n // NB,),
        in_specs=[pl.BlockSpec((NB, H), lambda i: (i, 0)),
                  pl.BlockSpec((NB, 128), lambda i: (i, 0)),
                  pl.BlockSpec((NB, 3 * H), lambda i: (i, 0)),
                  pl.BlockSpec((NSLICE, NB, SW), lambda i: (0, i, 0)),
                  wspec, wspec, wspec, wspec],
        out_specs=[pl.BlockSpec((NB, H), lambda i: (i, 0)),
                   pl.BlockSpec((NB, H), lambda i: (i, 0)),
                   pl.BlockSpec((NB, 3 * H), lambda i: (i, 0))],
        out_shape=[jax.ShapeDtypeStruct((n, H), jnp.float32),
                   jax.ShapeDtypeStruct((n, H), jnp.float32),
                   jax.ShapeDtypeStruct((n, 3 * H), jnp.float32)],
    )(h, du_p, v2, vupd6, lin_scalar_w, bs, lin_angular_w, ba)
    v_updated = v_updated.reshape(n, 3, H)

    return (h_updated, v_updated, f_updated, angular_info, dihedral_info, du)


# speedup vs baseline: 10.5788x; 1.0007x over previous
import functools

import jax
import jax.numpy as jnp
from jax import lax
from jax.experimental import pallas as pl
from jax.experimental.pallas import tpu as pltpu
from jax.experimental.pallas import tpu_sc as plsc

H = 256
R = 50
RP = 64
CUTOFF = 10.0

EB = 1000
NB = 1000

NSLICE = 6
SW = 128
TECS = 16
KB = 80


def _sc_pos_gather_call(pos8, rowi, coli):
    e = rowi.shape[0]
    NW = 2 * TECS
    per_w = e // NW
    K = 200
    nb = per_w // K
    mesh = plsc.VectorSubcoreMesh(core_axis_name="c", subcore_axis_name="s")

    @functools.partial(
        pl.kernel, mesh=mesh,
        out_type=[jax.ShapeDtypeStruct((e, 128), jnp.float32),
                  jax.ShapeDtypeStruct((e, 128), jnp.float32)],
        scratch_types=[
            pltpu.VMEM((K,), jnp.int32),
            pltpu.VMEM((K, 128), jnp.float32),
            pltpu.SemaphoreType.DMA,
        ])
    def body(pos_hbm, row_hbm, col_hbm, pr_hbm, pc_hbm, idx_v, st_v, sem):
        c = lax.axis_index("c")
        t = lax.axis_index("s")
        wbase = (t * 2 + c) * per_w

        def batch(b, _):
            base = wbase + b * K
            pltpu.sync_copy(row_hbm.at[pl.ds(base, K)], idx_v)
            pltpu.async_copy(pos_hbm.at[idx_v], st_v, sem).wait()
            pltpu.sync_copy(st_v, pr_hbm.at[pl.ds(base, K)])
            pltpu.sync_copy(col_hbm.at[pl.ds(base, K)], idx_v)
            pltpu.async_copy(pos_hbm.at[idx_v], st_v, sem).wait()
            pltpu.sync_copy(st_v, pc_hbm.at[pl.ds(base, K)])
            return 0

        lax.fori_loop(0, nb, batch, 0)

    return body(pos8, rowi, coli)


def _sc_hv_gather_call(h, v2, rowi, coli):
    e = rowi.shape[0]
    NW = 2 * TECS
    KH = 80
    nch = e // KH
    nrounds = (nch + NW - 1) // NW
    mesh = plsc.VectorSubcoreMesh(core_axis_name="c", subcore_axis_name="s")

    @functools.partial(
        pl.kernel, mesh=mesh,
        out_type=[jax.ShapeDtypeStruct((e, H), jnp.float32),
                  jax.ShapeDtypeStruct((e, H), jnp.float32),
                  jax.ShapeDtypeStruct((e, 3 * H), jnp.float32)],
        scratch_types=[
            pltpu.VMEM((KH,), jnp.int32),
            pltpu.VMEM((KH,), jnp.int32),
            pltpu.VMEM((KH, H), jnp.float32),
            pltpu.VMEM((KH, H), jnp.float32),
            pltpu.VMEM((KH, 3 * H), jnp.float32),
            pltpu.SemaphoreType.DMA,
            pltpu.SemaphoreType.DMA,
            pltpu.SemaphoreType.DMA,
            pltpu.SemaphoreType.DMA,
        ])
    def body(h_hbm, v_hbm, row_hbm, col_hbm, hc_hbm, hr_hbm, vr_hbm,
             idxc_v, idxr_v, shc_v, shr_v, sv_v, semg, semc, semr, semv):
        c = lax.axis_index("c")
        t = lax.axis_index("s")
        wid = t * 2 + c

        def batch(b, _):
            cid = b * NW + wid

            @pl.when(cid < nch)
            def _():
                base = cid * KH
                @pl.when(b > 0)
                def _():
                    pbase = (b - 1) * NW * KH + wid * KH
                    pltpu.make_async_copy(
                        shc_v, hc_hbm.at[pl.ds(pbase, KH)], semc).wait()
                    pltpu.make_async_copy(
                        shr_v, hr_hbm.at[pl.ds(pbase, KH)], semr).wait()
                    pltpu.make_async_copy(
                        sv_v, vr_hbm.at[pl.ds(pbase, KH)], semv).wait()

                pltpu.sync_copy(col_hbm.at[pl.ds(base, KH)], idxc_v)
                pltpu.sync_copy(row_hbm.at[pl.ds(base, KH)], idxr_v)
                pltpu.async_copy(h_hbm.at[idxc_v], shc_v, semg).wait()
                pltpu.async_copy(shc_v, hc_hbm.at[pl.ds(base, KH)], semc)
                pltpu.async_copy(h_hbm.at[idxr_v], shr_v, semg).wait()
                pltpu.async_copy(shr_v, hr_hbm.at[pl.ds(base, KH)], semr)
                pltpu.async_copy(v_hbm.at[idxr_v], sv_v, semg).wait()
                pltpu.async_copy(sv_v, vr_hbm.at[pl.ds(base, KH)], semv)
            return 0

        lax.fori_loop(0, nrounds, batch, 0)
        last_b = (nch - 1 - wid) // NW
        lbase = (last_b * NW + wid) * KH
        pltpu.make_async_copy(shc_v, hc_hbm.at[pl.ds(lbase, KH)], semc).wait()
        pltpu.make_async_copy(shr_v, hr_hbm.at[pl.ds(lbase, KH)], semr).wait()
        pltpu.make_async_copy(sv_v, vr_hbm.at[pl.ds(lbase, KH)], semv).wait()

    return body(h, v2, rowi, coli)


def _sc_du_scatter_call(pu, nu, rowi, coli, zeros_n8, n):
    e = rowi.shape[0]
    per_tec = e // TECS
    K = 80
    nb = per_tec // K
    RC = 80
    n_chunks = n // RC
    k_rounds = (n_chunks + TECS - 1) // TECS
    mesh = plsc.VectorSubcoreMesh(core_axis_name="c", subcore_axis_name="s")

    @functools.partial(
        pl.kernel, mesh=mesh,
        out_type=jax.ShapeDtypeStruct((n, 128), jnp.float32),
        scratch_types=[
            pltpu.VMEM((K,), jnp.int32),
            pltpu.VMEM((K,), jnp.int32),
            pltpu.VMEM((K,), jnp.int32),
            pltpu.VMEM((K,), jnp.int32),
            pltpu.VMEM((K, 128), jnp.float32),
            pltpu.VMEM((K, 128), jnp.float32),
            pltpu.VMEM((K, 128), jnp.float32),
            pltpu.VMEM((K, 128), jnp.float32),
            pltpu.VMEM_SHARED((n, 128), jnp.float32),
            pltpu.SemaphoreType.DMA,
            pltpu.SemaphoreType.DMA,
            pltpu.SemaphoreType.DMA,
            pltpu.SemaphoreType.DMA,
            pltpu.SemaphoreType.DMA,
            pltpu.SemaphoreType.DMA,
            pltpu.SemaphoreType.DMA,
            pltpu.SemaphoreType.DMA,
        ])
    def body(pu_hbm, nu_hbm, row_hbm, col_hbm, zero_hbm, out_hbm,
             idxr0, idxr1, idxc0, idxc1, str0, str1, stc0, stc1, acc_sh,
             sir0, sir1, sic0, sic1, svr0, svr1, svc0, svc1):
        c = lax.axis_index("c")
        t = lax.axis_index("s")
        idxr = (idxr0, idxr1)
        idxc = (idxc0, idxc1)
        strb = (str0, str1)
        stcb = (stc0, stc1)
        sir = (sir0, sir1)
        sic = (sic0, sic1)
        svr = (svr0, svr1)
        svc = (svc0, svc1)

        @pl.when(c == 0)
        def _():
            for k in range(k_rounds):
                cid = k * TECS + t

                @pl.when(cid < n_chunks)
                def _():
                    pltpu.sync_copy(zero_hbm.at[pl.ds(cid * RC, RC)],
                                    acc_sh.at[pl.ds(cid * RC, RC)])
            plsc.subcore_barrier()
            ebase = t * per_tec

            def load(b, j):
                base = ebase + b * K
                pltpu.async_copy(row_hbm.at[pl.ds(base, K)], idxr[j], sir[j])
                pltpu.async_copy(pu_hbm.at[pl.ds(base, K)], strb[j], svr[j])
                pltpu.async_copy(col_hbm.at[pl.ds(base, K)], idxc[j], sic[j])
                pltpu.async_copy(nu_hbm.at[pl.ds(base, K)], stcb[j], svc[j])

            def drain_scatter(b, j):
                base = ebase + b * K
                pltpu.make_async_copy(row_hbm.at[pl.ds(base, K)],
                                      idxr[j], sir[j]).wait()
                pltpu.make_async_copy(pu_hbm.at[pl.ds(base, K)],
                                      strb[j], svr[j]).wait()
                pltpu.sync_copy(strb[j], acc_sh.at[idxr[j]], add=True)
                pltpu.make_async_copy(col_hbm.at[pl.ds(base, K)],
                                      idxc[j], sic[j]).wait()
                pltpu.make_async_copy(nu_hbm.at[pl.ds(base, K)],
                                      stcb[j], svc[j]).wait()
                pltpu.sync_copy(stcb[j], acc_sh.at[idxc[j]], add=True)

            load(0, 0)

            def pair(k2, _):
                b0 = 2 * k2
                load(b0 + 1, 1)
                drain_scatter(b0, 0)

                @pl.when(b0 + 2 < nb)
                def _():
                    load(b0 + 2, 0)
                drain_scatter(b0 + 1, 1)
                return 0

            lax.fori_loop(0, nb // 2, pair, 0)
            if nb % 2 == 1:
                drain_scatter(nb - 1, 0)
            plsc.subcore_barrier()
            for k in range(k_rounds):
                cid = k * TECS + t

                @pl.when(cid < n_chunks)
                def _():
                    pltpu.sync_copy(acc_sh.at[pl.ds(cid * RC, RC)],
                                    out_hbm.at[pl.ds(cid * RC, RC)])

    return body(pu, nu, rowi, coli, zeros_n8)


def _sc_du_gather_call(du8, rowi, coli):
    e = rowi.shape[0]
    NW = 2 * TECS
    per_w = e // NW
    K = 200
    nb = per_w // K
    mesh = plsc.VectorSubcoreMesh(core_axis_name="c", subcore_axis_name="s")

    @functools.partial(
        pl.kernel, mesh=mesh,
        out_type=[jax.ShapeDtypeStruct((e, 128), jnp.float32),
                  jax.ShapeDtypeStruct((e, 128), jnp.float32)],
        scratch_types=[
            pltpu.VMEM((K,), jnp.int32),
            pltpu.VMEM((K, 128), jnp.float32),
            pltpu.SemaphoreType.DMA,
        ])
    def body(du_hbm, row_hbm, col_hbm, dur_hbm, duc_hbm, idx_v, st_v, sem):
        c = lax.axis_index("c")
        t = lax.axis_index("s")
        wbase = (t * 2 + c) * per_w

        def batch(b, _):
            base = wbase + b * K
            pltpu.sync_copy(row_hbm.at[pl.ds(base, K)], idx_v)
            pltpu.async_copy(du_hbm.at[idx_v], st_v, sem).wait()
            pltpu.sync_copy(st_v, dur_hbm.at[pl.ds(base, K)])
            pltpu.sync_copy(col_hbm.at[pl.ds(base, K)], idx_v)
            pltpu.async_copy(du_hbm.at[idx_v], st_v, sem).wait()
            pltpu.sync_copy(st_v, duc_hbm.at[pl.ds(base, K)])
            return 0

        lax.fori_loop(0, nb, batch, 0)

    return body(du8, rowi, coli)


def _sc_scatter_call(msg6, col, zeros_nw, n):
    e = col.shape[0]
    per_tec = e // TECS
    nb = per_tec // KB
    RC = 80
    n_chunks = n // RC
    k_rounds = (n_chunks + TECS - 1) // TECS
    mesh = plsc.VectorSubcoreMesh(core_axis_name="c", subcore_axis_name="s")

    @functools.partial(
        pl.kernel, mesh=mesh,
        out_type=jax.ShapeDtypeStruct((NSLICE, n, SW), jnp.float32),
        scratch_types=[
            pltpu.VMEM((KB,), jnp.int32),
            pltpu.VMEM((KB,), jnp.int32),
            pltpu.VMEM((KB, SW), jnp.float32),
            pltpu.VMEM((KB, SW), jnp.float32),
            pltpu.VMEM_SHARED((n, SW), jnp.float32),
            pltpu.SemaphoreType.DMA,
            pltpu.SemaphoreType.DMA,
            pltpu.SemaphoreType.DMA,
            pltpu.SemaphoreType.DMA,
        ])
    def body(msg_hbm, col_hbm, zero_hbm, out_hbm,
             idx0_v, idx1_v, st0_v, st1_v, acc_sh,
             semi0, semi1, semm0, semm1):
        c = lax.axis_index("c")
        t = lax.axis_index("s")
        ebase = t * per_tec
        idx_bufs = (idx0_v, idx1_v)
        st_bufs = (st0_v, st1_v)
        sems_i = (semi0, semi1)
        sems_m = (semm0, semm1)

        def load(b, j):
            base = ebase + b * KB
            pltpu.async_copy(col_hbm.at[pl.ds(base, KB)], idx_bufs[j],
                             sems_i[j])
            pltpu.async_copy(msg_hbm.at[s, pl.ds(base, KB)], st_bufs[j],
                             sems_m[j])

        def drain_scatter(b, j):
            base = ebase + b * KB
            pltpu.make_async_copy(col_hbm.at[pl.ds(base, KB)],
                                  idx_bufs[j], sems_i[j]).wait()
            pltpu.make_async_copy(msg_hbm.at[s, pl.ds(base, KB)],
                                  st_bufs[j], sems_m[j]).wait()
            pltpu.sync_copy(st_bufs[j], acc_sh.at[idx_bufs[j]], add=True)

        for s_local in range(NSLICE // 2):
            s = c * (NSLICE // 2) + s_local
            for k in range(k_rounds):
                cid = k * TECS + t

                @pl.when(cid < n_chunks)
                def _():
                    pltpu.sync_copy(zero_hbm.at[pl.ds(cid * RC, RC)],
                                    acc_sh.at[pl.ds(cid * RC, RC)])
            plsc.subcore_barrier()

            load(0, 0)

            def pair(k2, _):
                b0 = 2 * k2
                load(b0 + 1, 1)
                drain_scatter(b0, 0)

                @pl.when(b0 + 2 < nb)
                def _():
                    load(b0 + 2, 0)
                drain_scatter(b0 + 1, 1)
                return 0

            lax.fori_loop(0, nb // 2, pair, 0)
            if nb % 2 == 1:
                drain_scatter(nb - 1, 0)
            plsc.subcore_barrier()
            for k in range(k_rounds):
                cid = k * TECS + t

                @pl.when(cid < n_chunks)
                def _():
                    pltpu.sync_copy(acc_sh.at[pl.ds(cid * RC, RC)],
                                    out_hbm.at[s, pl.ds(cid * RC, RC)])
            plsc.subcore_barrier()

    return body(msg6, col, zeros_nw)


def _geom_body(pc_ref, pr_ref, geom_ref, negu_ref):
    pc = pc_ref[...]
    pr = pr_ref[...]
    ev = pc - pr
    e0 = ev[:, 0:1]
    e1 = ev[:, 1:2]
    e2 = ev[:, 2:3]
    dist = jnp.sqrt(e0 * e0 + e1 * e1 + e2 * e2) + 1e-8
    inv = 1.0 / dist
    u0 = e0 * inv
    u1 = e1 * inv
    u2 = e2 * inv
    cw = 0.5 * (jnp.cos(jnp.pi * dist / CUTOFF) + 1.0)
    cw = jnp.where(dist < CUTOFF, cw, 0.0)
    zero = jnp.zeros((u0.shape[0], 124), jnp.float32)
    geom_ref[...] = jnp.concatenate([u0, u1, u2, cw, zero], axis=1)
    negu_ref[...] = jnp.concatenate(
        [-u0, -u1, -u2, jnp.zeros_like(u0), zero], axis=1)


def _msg_body(hc_ref, hr_ref, rbf_ref, vr_ref, geom_ref,
              w1a_ref, w1b_ref, w1c_ref, b1_ref, wv_ref, bv_ref,
              msg_ref):
    hc = hc_ref[...]
    hr = hr_ref[...]
    rbf = rbf_ref[...]
    geom = geom_ref[...]
    u0 = geom[:, 0:1]
    u1 = geom[:, 1:2]
    u2 = geom[:, 2:3]
    cw = geom[:, 3:4]

    sm = (jnp.dot(hc, w1a_ref[...], preferred_element_type=jnp.float32)
          + jnp.dot(hr, w1b_ref[...], preferred_element_type=jnp.float32)
          + jnp.dot(rbf, w1c_ref[...], preferred_element_type=jnp.float32)
          + b1_ref[...])
    vw = jnp.dot(sm, wv_ref[...], preferred_element_type=jnp.float32) + bv_ref[...]
    w1 = vw[:, :H] * cw
    w2 = vw[:, H:] * cw
    vr = vr_ref[...]
    us = (u0, u1, u2)
    for s in range(NSLICE):
        d, half = s // 2, s % 2
        msg_ref[s] = (w1[:, SW * half:SW * (half + 1)] * us[d]
                      + w2[:, SW * half:SW * (half + 1)]
                      * vr[:, H * d + SW * half:H * d + SW * (half + 1)])


def _fup_body(f_ref, geom_ref, dur_ref, duc_ref,
              we_ref, be_ref, wd_ref, bd_ref,
              fout_ref, dih_ref):
    geom = geom_ref[...]
    u0 = geom[:, 0:1]
    u1 = geom[:, 1:2]
    u2 = geom[:, 2:3]
    dur = dur_ref[...]
    duc = duc_ref[...]
    a0 = dur[:, 0:1]
    a1 = dur[:, 1:2]
    a2 = dur[:, 2:3]
    b0 = duc[:, 0:1]
    b1 = duc[:, 1:2]
    b2 = duc[:, 2:3]
    dvi = a0 * u0 + a1 * u1 + a2 * u2
    dvj = -(b0 * u0 + b1 * u1 + b2 * u2)
    w_ij0 = a0 - dvi * u0
    w_ij1 = a1 - dvi * u1
    w_ij2 = a2 - dvi * u2
    w_ji0 = b0 + dvj * u0
    w_ji1 = b1 + dvj * u1
    w_ji2 = b2 + dvj * u2
    dih = w_ij0 * w_ji0 + w_ij1 * w_ji1 + w_ij2 * w_ji2
    dih_ref[...] = jnp.broadcast_to(dih, (dih.shape[0], H))

    wd_bf = wd_ref[...].astype(jnp.bfloat16).astype(jnp.float32)
    colsum_d = jnp.sum(wd_bf, axis=0, keepdims=True)
    dih_bf = dih.astype(jnp.bfloat16).astype(jnp.float32)
    dmod = jax.nn.sigmoid(dih_bf * colsum_d + bd_ref[...])
    f = f_ref[...]
    fout_ref[...] = f + (jnp.dot(f, we_ref[...], preferred_element_type=jnp.float32)
                         + be_ref[...]) * dmod


def _node_body(h_ref, du_ref, v_ref, vu_ref, ws_ref, bs_ref, wa_ref, ba_ref,
               hout_ref, ang_ref, vout_ref):
    du = du_ref[...]
    ang = du[:, 0:1] ** 2 + du[:, 1:2] ** 2 + du[:, 2:3] ** 2
    ang_ref[...] = jnp.broadcast_to(ang, (ang.shape[0], H))
    wa_bf = wa_ref[...].astype(jnp.bfloat16).astype(jnp.float32)
    colsum_a = jnp.sum(wa_bf, axis=0, keepdims=True)
    ang_bf = ang.astype(jnp.bfloat16).astype(jnp.float32)
    amod = jax.nn.sigmoid(ang_bf * colsum_a + ba_ref[...])
    h = h_ref[...]
    hout_ref[...] = h + (jnp.dot(h, ws_ref[...], preferred_element_type=jnp.float32)
                         + bs_ref[...]) * amod
    v = v_ref[...]
    for s in range(NSLICE):
        vout_ref[:, SW * s:SW * (s + 1)] = v[:, SW * s:SW * (s + 1)] + vu_ref[s]


def kernel(h, v, f, pos, edge_index, edge_rbf,
           lin_msg_w, lin_msg_b, lin_vec_w, lin_vec_b,
           lin_scalar_w, lin_scalar_b, lin_edge_w, lin_edge_b,
           lin_angular_w, lin_angular_b, lin_dihedral_w, lin_dihedral_b):
    n = pos.shape[0]
    e = edge_index.shape[1]
    row = edge_index[0]
    col = edge_index[1]

    pos8 = jnp.pad(pos, ((0, 0), (0, 125)))
    pr8, pc8 = _sc_pos_gather_call(pos8, row, col)
    v2 = v.reshape(n, 3 * H)
    hc, hr, vr = _sc_hv_gather_call(h, v2, row, col)

    geom, negu = pl.pallas_call(
        _geom_body,
        grid=(e // EB,),
        in_specs=[pl.BlockSpec((EB, 128), lambda i: (i, 0)),
                  pl.BlockSpec((EB, 128), lambda i: (i, 0))],
        out_specs=[pl.BlockSpec((EB, 128), lambda i: (i, 0)),
                   pl.BlockSpec((EB, 128), lambda i: (i, 0))],
        out_shape=[jax.ShapeDtypeStruct((e, 128), jnp.float32),
                   jax.ShapeDtypeStruct((e, 128), jnp.float32)],
    )(pc8, pr8)

    zeros_n8 = jnp.zeros((n, 128), jnp.float32)
    du8 = _sc_du_scatter_call(geom, negu, row, col, zeros_n8, n)
    dur8, duc8 = _sc_du_gather_call(du8, row, col)
    du = du8[:, :3]

    rbf_p = jnp.pad(edge_rbf, ((0, 0), (0, RP - R)))
    w1a = lin_msg_w[:H]
    w1b = lin_msg_w[H:2 * H]
    w1c = jnp.pad(lin_msg_w[2 * H:], ((0, RP - R), (0, 0)))
    b1 = lin_msg_b.reshape(1, H)
    bv = lin_vec_b.reshape(1, 2 * H)
    be = lin_edge_b.reshape(1, H)
    bd = lin_dihedral_b.reshape(1, H)
    bs = lin_scalar_b.reshape(1, H)
    ba = lin_angular_b.reshape(1, H)

    wspec = pl.BlockSpec(None, lambda i: (0, 0))
    espec = lambda w: pl.BlockSpec((EB, w), lambda i: (i, 0))
    msg6 = pl.pallas_call(
        _msg_body,
        grid=(e // EB,),
        in_specs=[espec(H), espec(H), espec(RP), espec(3 * H),
                  espec(128),
                  wspec, wspec, wspec, wspec, wspec, wspec],
        out_specs=pl.BlockSpec((NSLICE, EB, SW), lambda i: (0, i, 0)),
        out_shape=jax.ShapeDtypeStruct((NSLICE, e, SW), jnp.float32),
    )(hc, hr, rbf_p, vr, geom,
      w1a, w1b, w1c, b1, lin_vec_w, bv)

    f_updated, dihedral_info = pl.pallas_call(
        _fup_body,
        grid=(e // EB,),
        in_specs=[espec(H), espec(128), espec(128), espec(128),
                  wspec, wspec, wspec, wspec],
        out_specs=[espec(H), espec(H)],
        out_shape=[jax.ShapeDtypeStruct((e, H), jnp.float32),
                   jax.ShapeDtypeStruct((e, H), jnp.float32)],
    )(f, geom, dur8, duc8, lin_edge_w, be, lin_dihedral_w, bd)

    zeros_nw = jnp.zeros((n, SW), jnp.float32)
    vupd6 = _sc_scatter_call(msg6, col, zeros_nw, n)

    du_p = du8
    h_updated, angular_info, v_updated = pl.pallas_call(
        _node_body,
        grid=(n // NB,),
        in_specs=[pl.BlockSpec((NB, H), lambda i: (i, 0)),
                  pl.BlockSpec((NB, 128), lambda i: (i, 0)),
                  pl.BlockSpec((NB, 3 * H), lambda i: (i, 0)),
                  pl.BlockSpec((NSLICE, NB, SW), lambda i: (0, i, 0)),
                  wspec, wspec, wspec, wspec],
        out_specs=[pl.BlockSpec((NB, H), lambda i: (i, 0)),
                   pl.BlockSpec((NB, H), lambda i: (i, 0)),
                   pl.BlockSpec((NB, 3 * H), lambda i: (i, 0))],
        out_shape=[jax.ShapeDtypeStruct((n, H), jnp.float32),
                   jax.ShapeDtypeStruct((n, H), jnp.float32),
                   jax.ShapeDtypeStruct((n, 3 * H), jnp.float32)],
    )(h, du_p, v2, vupd6, lin_scalar_w, bs, lin_angular_w, ba)
    v_updated = v_updated.reshape(n, 3, H)

    return (h_updated, v_updated, f_updated, angular_info, dihedral_info, du)
